# baseline (speedup 1.0000x reference)
import functools

import jax
import jax.numpy as jnp
from jax.experimental import pallas as pl
from jax.experimental.pallas import tpu as pltpu
from jax.experimental.pallas import tpu_sc as plsc

_N_CENTER = 512
_N_NEAR = 32
_B = 8
_N = 2048
_SBLK = 16
_NSB = _N_CENTER // _SBLK
_BR = 1024
_ROWS = _B * _N_CENTER * _N_NEAR
_CROWS = _B * _N_CENTER
_F_CH = 76
_IN_CH = 82
_EPS = 1e-5


def _dot(a, b):
    return jax.lax.dot_general(a, b, (((1,), (0,)), ((), ())),
                               preferred_element_type=jnp.float32)


def _dotb(a, b):
    return jax.lax.dot_general(a.astype(jnp.bfloat16), b.astype(jnp.bfloat16),
                               (((1,), (0,)), ((), ())),
                               preferred_element_type=jnp.float32)


def _dot_t(a, b):
    return jax.lax.dot_general(a, b, (((1,), (1,)), ((), ())),
                               preferred_element_type=jnp.float32)


def _fps_body(xs_ref, ys_ref, zs_ref, f0_ref, out_ref):
    xs = xs_ref[...]
    ys = ys_ref[...]
    zs = zs_ref[...]
    col = jax.lax.broadcasted_iota(jnp.int32, (_B, _N), 1)
    lane_c = jax.lax.broadcasted_iota(jnp.int32, (_B, _N_CENTER), 1)

    sxyz = jnp.concatenate([xs, ys, zs], axis=0)
    col3 = jax.lax.broadcasted_iota(jnp.int32, (3 * _B, _N), 1)

    def step(t, carry):
        dist, far, cents = carry
        cents = jnp.where(lane_c == t, far, cents)
        far3 = jnp.concatenate([far, far, far], axis=0)
        cs = jnp.sum(jnp.where(col3 == far3, sxyz, 0.0), axis=1,
                     keepdims=True)
        dx = xs - cs[0:_B]
        dy = ys - cs[_B:2 * _B]
        dz = zs - cs[2 * _B:3 * _B]
        d = (dx * dx + dy * dy) + dz * dz
        dist = jnp.minimum(dist, d)
        m = jnp.max(dist, axis=1, keepdims=True)
        far = jnp.min(jnp.where(dist == m, col, _N), axis=1, keepdims=True)
        return dist, far, cents

    dist0 = jnp.full((_B, _N), 1e10, jnp.float32)
    far0 = f0_ref[...]
    cents0 = jnp.zeros((_B, _N_CENTER), jnp.int32)
    _, _, cents = jax.lax.fori_loop(0, _N_CENTER, step, (dist0, far0, cents0))
    out_ref[...] = cents


def _run_fps(xs, ys, zs, f0):
    return pl.pallas_call(
        _fps_body,
        out_shape=jax.ShapeDtypeStruct((_B, _N_CENTER), jnp.int32),
    )(xs, ys, zs, f0)


def _knn_body(p_ref, xs_ref, ys_ref, zs_ref, fps_ref, out_ref):
    p = p_ref[0]
    xs = xs_ref[0]
    ys = ys_ref[0]
    zs = zs_ref[0]
    fpsc = fps_ref[0]

    colc = jax.lax.broadcasted_iota(jnp.int32, (_N_CENTER, _N), 1)
    oh = (colc == fpsc).astype(jnp.float32)
    px = p[:, 0:1]
    py = p[:, 1:2]
    pz = p[:, 2:3]
    sqf = (px * px + py * py) + pz * pz
    aug = jnp.concatenate([p, sqf], axis=1)
    cg = _dot(oh, aug)
    c = cg[:, 0:3]
    sqc = cg[:, 3:4]
    sq_row = (xs * xs + ys * ys) + zs * zs
    dots = _dot_t(c, p)
    d = (sqc + sq_row) - 2.0 * dots

    lane_k = jax.lax.broadcasted_iota(jnp.int32, (_N_CENTER, _N_NEAR), 1)

    def step(k, carry):
        d_cur, acc = carry
        m = jnp.min(d_cur, axis=1, keepdims=True)
        eq = d_cur == m
        cand = jnp.where(eq, colc, _N)
        j = jnp.min(cand, axis=1, keepdims=True)
        acc = jnp.where(lane_k == k, j, acc)
        d_cur = jnp.where(cand == j, jnp.inf, d_cur)
        return d_cur, acc

    acc0 = jnp.zeros((_N_CENTER, _N_NEAR), jnp.int32)
    _, acc = jax.lax.fori_loop(0, _N_NEAR, step, (d, acc0))
    out_ref[0] = acc


def _run_knn(xyz, xs, ys, zs, fps3):
    return pl.pallas_call(
        _knn_body,
        grid=(_B,),
        in_specs=[
            pl.BlockSpec((1, _N, 3), lambda b: (b, 0, 0)),
            pl.BlockSpec((1, 1, _N), lambda b: (b, 0, 0)),
            pl.BlockSpec((1, 1, _N), lambda b: (b, 0, 0)),
            pl.BlockSpec((1, 1, _N), lambda b: (b, 0, 0)),
            pl.BlockSpec((1, _N_CENTER, 1), lambda b: (b, 0, 0)),
        ],
        out_specs=pl.BlockSpec((1, _N_CENTER, _N_NEAR), lambda b: (b, 0, 0)),
        out_shape=jax.ShapeDtypeStruct((_B, _N_CENTER, _N_NEAR), jnp.int32),
    )(xyz, xs, ys, zs, fps3)


_GROWS = _ROWS + _CROWS
_NW = 32
_RPW = _GROWS // _NW
_GCH = 6
_CHUNK = _RPW // _GCH
_PCH = 128


def _run_sc_gather(table, gidx):
    mesh = plsc.VectorSubcoreMesh(core_axis_name="c", subcore_axis_name="s")

    @functools.partial(
        pl.kernel, mesh=mesh,
        out_type=jax.ShapeDtypeStruct((_GROWS, _PCH), jnp.float32),
        scratch_types=[
            pltpu.VMEM((_CHUNK,), jnp.int32),
            pltpu.VMEM((_CHUNK, _PCH), jnp.float32),
            pltpu.SemaphoreType.DMA,
        ],
    )
    def k(table_hbm, idx_hbm, out_hbm, idx_v, rows_v, sem):
        wid = jax.lax.axis_index("s") * 2 + jax.lax.axis_index("c")
        base = wid * _RPW

        def body(i, carry):
            off = base + i * _CHUNK
            pltpu.sync_copy(idx_hbm.at[pl.ds(off, _CHUNK)], idx_v)
            pltpu.async_copy(table_hbm.at[idx_v], rows_v, sem).wait()
            pltpu.sync_copy(rows_v, out_hbm.at[pl.ds(off, _CHUNK)])
            return carry

        jax.lax.fori_loop(0, _GCH, body, 0)

    return k(table, gidx)


def _l1_body(xg_ref, cg_ref, w_ref, b_ref, o_ref, s1_ref, s2_ref):
    xg = xg_ref[0, 0]
    cg = cg_ref[0, 0]

    cb = jnp.broadcast_to(cg.reshape(_SBLK, 1, _PCH),
                          (_SBLK, _N_NEAR, _PCH)).reshape(
                              _SBLK * _N_NEAR, _PCH)
    x0 = jnp.concatenate([
        xg[:, 0:3] - cb[:, 0:3],
        xg[:, 3:6] - cb[:, 3:6],
        xg[:, 6:8], cb[:, 6:8],
        xg[:, 8:12], cb[:, 8:12],
        xg[:, 12:76],
    ], axis=1)
    p = _dotb(x0, w_ref[...]) + b_ref[...]
    o_ref[0, 0] = p.astype(jnp.bfloat16)
    s1_ref[0, 0] = jnp.sum(p, axis=0, keepdims=True)
    s2_ref[0, 0] = jnp.sum(p * p, axis=0, keepdims=True)


def _run_l1(xg4, cg4, w, b2):
    cout = w.shape[1]
    outs = pl.pallas_call(
        _l1_body,
        grid=(_B, _NSB),
        in_specs=[
            pl.BlockSpec((1, 1, _SBLK * _N_NEAR, _PCH),
                         lambda b, s: (b, s, 0, 0)),
            pl.BlockSpec((1, 1, _SBLK, _PCH), lambda b, s: (b, s, 0, 0)),
            pl.BlockSpec((_IN_CH, cout), lambda b, s: (0, 0)),
            pl.BlockSpec((1, cout), lambda b, s: (0, 0)),
        ],
        out_specs=[
            pl.BlockSpec((1, 1, _SBLK * _N_NEAR, cout),
                         lambda b, s: (b, s, 0, 0)),
            pl.BlockSpec((1, 1, 1, cout), lambda b, s: (b, s, 0, 0)),
            pl.BlockSpec((1, 1, 1, cout), lambda b, s: (b, s, 0, 0)),
        ],
        out_shape=[
            jax.ShapeDtypeStruct((_B, _NSB, _SBLK * _N_NEAR, cout),
                                 jnp.bfloat16),
            jax.ShapeDtypeStruct((_B, _NSB, 1, cout), jnp.float32),
            jax.ShapeDtypeStruct((_B, _NSB, 1, cout), jnp.float32),
        ],
    )(xg4, cg4, w, b2)
    p, s1, s2 = outs
    p = p.reshape(_ROWS, cout)
    mean = jnp.sum(s1, axis=(0, 1, 2)) / _ROWS
    var = jnp.sum(s2, axis=(0, 1, 2)) / _ROWS - mean * mean
    return p, mean, var


def _mm_body(aff, relu, stats, x_ref, a_ref, c_ref, w_ref, b_ref,
             o_ref, s1_ref, s2_ref):
    h = x_ref[...].astype(jnp.float32)
    if aff:
        h = h * a_ref[...] + c_ref[...]
    if relu:
        h = jnp.maximum(h, 0.0)
    p = _dotb(h, w_ref[...]) + b_ref[...]
    o_ref[...] = p.astype(jnp.bfloat16)
    if stats:
        s1_ref[0] = jnp.sum(p, axis=0, keepdims=True)
        s2_ref[0] = jnp.sum(p * p, axis=0, keepdims=True)


def _run_mm(x, w, b2, a2=None, c2=None, relu=False, stats=True):
    rows, cin = x.shape
    cout = w.shape[1]
    g = rows // _BR
    aff = a2 is not None
    if not aff:
        a2 = jnp.zeros((1, cin), jnp.float32)
        c2 = jnp.zeros((1, cin), jnp.float32)
    body = functools.partial(_mm_body, aff, relu, stats)
    outs = pl.pallas_call(
        body,
        grid=(g,),
        in_specs=[
            pl.BlockSpec((_BR, cin), lambda i: (i, 0)),
            pl.BlockSpec((1, cin), lambda i: (0, 0)),
            pl.BlockSpec((1, cin), lambda i: (0, 0)),
            pl.BlockSpec((cin, cout), lambda i: (0, 0)),
            pl.BlockSpec((1, cout), lambda i: (0, 0)),
        ],
        out_specs=[
            pl.BlockSpec((_BR, cout), lambda i: (i, 0)),
            pl.BlockSpec((1, 1, cout), lambda i: (i, 0, 0)),
            pl.BlockSpec((1, 1, cout), lambda i: (i, 0, 0)),
        ],
        out_shape=[
            jax.ShapeDtypeStruct((rows, cout), jnp.bfloat16),
            jax.ShapeDtypeStruct((g, 1, cout), jnp.float32),
            jax.ShapeDtypeStruct((g, 1, cout), jnp.float32),
        ],
    )(x, a2, c2, w, b2)
    p, s1, s2 = outs
    if not stats:
        return p, None, None
    mean = jnp.sum(s1, axis=(0, 1)) / rows
    var = jnp.sum(s2, axis=(0, 1)) / rows - mean * mean
    return p, mean, var


def _mm2_body(x_ref, a_ref, c_ref, w1_ref, b1_ref, w2_ref, b2_ref,
              o1_ref, o2_ref, s11_ref, s12_ref, s21_ref, s22_ref):
    h = jnp.maximum(x_ref[...].astype(jnp.float32) * a_ref[...] +
                    c_ref[...], 0.0)
    p1 = _dotb(h, w1_ref[...]) + b1_ref[...]
    p2 = _dotb(h, w2_ref[...]) + b2_ref[...]
    o1_ref[...] = p1.astype(jnp.bfloat16)
    o2_ref[...] = p2.astype(jnp.bfloat16)
    s11_ref[0] = jnp.sum(p1, axis=0, keepdims=True)
    s12_ref[0] = jnp.sum(p1 * p1, axis=0, keepdims=True)
    s21_ref[0] = jnp.sum(p2, axis=0, keepdims=True)
    s22_ref[0] = jnp.sum(p2 * p2, axis=0, keepdims=True)


def _run_mm2(x, a2, c2, w1, b1, w2, b2):
    rows, cin = x.shape
    cout = w1.shape[1]
    g = rows // _BR
    bcast = lambda i: (0, 0)
    outs = pl.pallas_call(
        _mm2_body,
        grid=(g,),
        in_specs=[
            pl.BlockSpec((_BR, cin), lambda i: (i, 0)),
            pl.BlockSpec((1, cin), bcast),
            pl.BlockSpec((1, cin), bcast),
            pl.BlockSpec((cin, cout), bcast),
            pl.BlockSpec((1, cout), bcast),
            pl.BlockSpec((cin, cout), bcast),
            pl.BlockSpec((1, cout), bcast),
        ],
        out_specs=[
            pl.BlockSpec((_BR, cout), lambda i: (i, 0)),
            pl.BlockSpec((_BR, cout), lambda i: (i, 0)),
            pl.BlockSpec((1, 1, cout), lambda i: (i, 0, 0)),
            pl.BlockSpec((1, 1, cout), lambda i: (i, 0, 0)),
            pl.BlockSpec((1, 1, cout), lambda i: (i, 0, 0)),
            pl.BlockSpec((1, 1, cout), lambda i: (i, 0, 0)),
        ],
        out_shape=[
            jax.ShapeDtypeStruct((rows, cout), jnp.bfloat16),
            jax.ShapeDtypeStruct((rows, cout), jnp.bfloat16),
            jax.ShapeDtypeStruct((g, 1, cout), jnp.float32),
            jax.ShapeDtypeStruct((g, 1, cout), jnp.float32),
            jax.ShapeDtypeStruct((g, 1, cout), jnp.float32),
            jax.ShapeDtypeStruct((g, 1, cout), jnp.float32),
        ],
    )(x, a2, c2, w1, b1, w2, b2)
    p1, p2, s11, s12, s21, s22 = outs
    m1 = jnp.sum(s11, axis=(0, 1)) / rows
    v1 = jnp.sum(s12, axis=(0, 1)) / rows - m1 * m1
    m2 = jnp.sum(s21, axis=(0, 1)) / rows
    v2 = jnp.sum(s22, axis=(0, 1)) / rows - m2 * m2
    return p1, m1, v1, p2, m2, v2


def _pag_body(pp_ref, ap_ref, cp_ref, wp2_ref, bp2_ref,
              pa_ref, aa_ref, ca_ref, wa2_ref, ba2_ref,
              fai_ref, wg1_ref, bg1_ref,
              al_ref, pg_ref, s1_ref, s2_ref):
    hp = jnp.maximum(pp_ref[...].astype(jnp.float32) * ap_ref[...] +
                     cp_ref[...], 0.0)
    psi2 = _dotb(hp, wp2_ref[...]) + bp2_ref[...]
    ha = jnp.maximum(pa_ref[...].astype(jnp.float32) * aa_ref[...] +
                     ca_ref[...], 0.0)
    al_ref[...] = (_dotb(ha, wa2_ref[...]) + ba2_ref[...]).astype(jnp.bfloat16)
    gc = _BR // _N_NEAR
    gin = (fai_ref[...].reshape(gc, 1, 256) -
           psi2.reshape(gc, _N_NEAR, 256)).reshape(_BR, 256)
    pg = _dotb(gin, wg1_ref[...]) + bg1_ref[...]
    pg_ref[...] = pg.astype(jnp.bfloat16)
    s1_ref[0] = jnp.sum(pg, axis=0, keepdims=True)
    s2_ref[0] = jnp.sum(pg * pg, axis=0, keepdims=True)


def _run_pag(pp1, ap, cp, wp2, bp2, pa1, aa, ca, wa2, ba2, fai2, wg1, bg1):
    g = _ROWS // _BR
    gc = _BR // _N_NEAR
    bcast2 = lambda i: (0, 0)
    outs = pl.pallas_call(
        _pag_body,
        grid=(g,),
        in_specs=[
            pl.BlockSpec((_BR, 264), lambda i: (i, 0)),
            pl.BlockSpec((1, 264), bcast2),
            pl.BlockSpec((1, 264), bcast2),
            pl.BlockSpec((264, 256), bcast2),
            pl.BlockSpec((1, 256), bcast2),
            pl.BlockSpec((_BR, 264), lambda i: (i, 0)),
            pl.BlockSpec((1, 264), bcast2),
            pl.BlockSpec((1, 264), bcast2),
            pl.BlockSpec((264, 256), bcast2),
            pl.BlockSpec((1, 256), bcast2),
            pl.BlockSpec((gc, 256), lambda i: (i, 0)),
            pl.BlockSpec((256, 264), bcast2),
            pl.BlockSpec((1, 264), bcast2),
        ],
        out_specs=[
            pl.BlockSpec((_BR, 256), lambda i: (i, 0)),
            pl.BlockSpec((_BR, 264), lambda i: (i, 0)),
            pl.BlockSpec((1, 1, 264), lambda i: (i, 0, 0)),
            pl.BlockSpec((1, 1, 264), lambda i: (i, 0, 0)),
        ],
        out_shape=[
            jax.ShapeDtypeStruct((_ROWS, 256), jnp.bfloat16),
            jax.ShapeDtypeStruct((_ROWS, 264), jnp.bfloat16),
            jax.ShapeDtypeStruct((g, 1, 264), jnp.float32),
            jax.ShapeDtypeStruct((g, 1, 264), jnp.float32),
        ],
    )(pp1, ap, cp, wp2, bp2, pa1, aa, ca, wa2, ba2, fai2, wg1, bg1)
    alpha2, pg1, s1, s2 = outs
    mean = jnp.sum(s1, axis=(0, 1)) / _ROWS
    var = jnp.sum(s2, axis=(0, 1)) / _ROWS - mean * mean
    return alpha2, pg1, mean, var


def _bn_affine(mean, var, g, be):
    a = g / jnp.sqrt(var + _EPS)
    c = be - mean * a
    return a.reshape(1, -1), c.reshape(1, -1)


def _cfa_body(cg_ref, w1_ref, b1_ref, g1_ref, e1_ref, w2_ref, b2_ref,
              g2_ref, e2_ref, w3_ref, b3_ref, g3_ref, e3_ref,
              wf1_ref, bf1_ref, gf1_ref, ef1_ref, wf2_ref, bf2_ref,
              out_ref):
    cg = cg_ref[...]
    x = jnp.concatenate([
        jnp.zeros((_CROWS, 6), jnp.float32),
        cg[:, 6:8], cg[:, 6:8],
        cg[:, 8:12], cg[:, 8:12],
        cg[:, 12:76],
    ], axis=1)

    def bnrelu(p, g, e):
        m = jnp.mean(p, axis=0, keepdims=True)
        v = jnp.mean((p - m) * (p - m), axis=0, keepdims=True)
        return jnp.maximum(g * (p - m) / jnp.sqrt(v + _EPS) + e, 0.0)

    x = bnrelu(_dotb(x, w1_ref[...]) + b1_ref[...], g1_ref[...], e1_ref[...])
    x = bnrelu(_dotb(x, w2_ref[...]) + b2_ref[...], g2_ref[...], e2_ref[...])
    x = bnrelu(_dotb(x, w3_ref[...]) + b3_ref[...], g3_ref[...], e3_ref[...])
    x = bnrelu(_dotb(x, wf1_ref[...]) + bf1_ref[...], gf1_ref[...],
               ef1_ref[...])
    out_ref[...] = _dotb(x, wf2_ref[...]) + bf2_ref[...]


def _run_cfa(cg_all, args):
    return pl.pallas_call(
        _cfa_body,
        out_shape=jax.ShapeDtypeStruct((_CROWS, 256), jnp.float32),
    )(cg_all, *args)


def _final_body(pg_ref, a_ref, c_ref, w_ref, b_ref, al_ref, y_ref):
    h = jnp.maximum(pg_ref[...].astype(jnp.float32) * a_ref[...] +
                    c_ref[...], 0.0)
    gam = _dotb(h, w_ref[...]) + b_ref[...]
    m = jnp.max(gam, axis=1, keepdims=True)
    e = jnp.exp(gam - m)
    s = jnp.sum(e, axis=1, keepdims=True)
    contrib = (e / s) * al_ref[...].astype(jnp.float32) * (256.0 / _N_NEAR)
    gc = _BR // _N_NEAR
    y_ref[...] = jnp.sum(contrib.reshape(gc, _N_NEAR, 256), axis=1)


def _run_final(pg1, ag, cg, w, b2, alpha2):
    g = _ROWS // _BR
    gc = _BR // _N_NEAR
    return pl.pallas_call(
        _final_body,
        grid=(g,),
        in_specs=[
            pl.BlockSpec((_BR, 264), lambda i: (i, 0)),
            pl.BlockSpec((1, 264), lambda i: (0, 0)),
            pl.BlockSpec((1, 264), lambda i: (0, 0)),
            pl.BlockSpec((264, 256), lambda i: (0, 0)),
            pl.BlockSpec((1, 256), lambda i: (0, 0)),
            pl.BlockSpec((_BR, 256), lambda i: (i, 0)),
        ],
        out_specs=pl.BlockSpec((gc, 256), lambda i: (i, 0)),
        out_shape=jax.ShapeDtypeStruct((_CROWS, 256), jnp.float32),
    )(pg1, ag, cg, w, b2, alpha2)


@jax.jit
def kernel(xyz, eula_angle, edge_nearby, meta_type, fea, params):
    xs = xyz[:, :, 0]
    ys = xyz[:, :, 1]
    zs = xyz[:, :, 2]
    f0 = jax.random.randint(jax.random.key(42), (_B,), 0, _N,
                            dtype=jnp.int32).reshape(_B, 1)

    fps_idx = _run_fps(xs, ys, zs, f0)
    fps3 = fps_idx.reshape(_B, _N_CENTER, 1)
    xs3 = xs.reshape(_B, 1, _N)
    ys3 = ys.reshape(_B, 1, _N)
    zs3 = zs.reshape(_B, 1, _N)
    idx = _run_knn(xyz, xs3, ys3, zs3, fps3)

    feat = jnp.concatenate([xyz, eula_angle, edge_nearby, meta_type, fea,
                            jnp.zeros((_B, _N, _PCH - _F_CH), jnp.float32)],
                           axis=2)
    boff = (jnp.arange(_B, dtype=jnp.int32) * _N).reshape(_B, 1, 1)
    gidx = jnp.concatenate([
        (idx + boff).reshape(-1),
        (fps_idx + boff[:, :, 0]).reshape(-1),
    ])
    gath = _run_sc_gather(feat.reshape(_B * _N, _PCH), gidx)
    xg4 = gath[:_ROWS].reshape(_B, _NSB, _SBLK * _N_NEAR, _PCH)
    cg_all = gath[_ROWS:]
    cg4 = cg_all.reshape(_B, _NSB, _SBLK, _PCH)
    cc = cg_all.reshape(_B, _N_CENTER, _PCH)

    mlp = params['mlp']
    att = params['att']
    wt = [jnp.transpose(layer['w']) for layer in mlp]
    bt = [layer['b'].reshape(1, -1) for layer in mlp]

    p1, m1, v1 = _run_l1(xg4, cg4, wt[0], bt[0])
    a1, c1 = _bn_affine(m1, v1, mlp[0]['g'], mlp[0]['be'])
    p2, m2, v2 = _run_mm(p1, wt[1], bt[1], a1, c1, relu=True)
    a2, c2 = _bn_affine(m2, v2, mlp[1]['g'], mlp[1]['be'])
    p3, m3, v3 = _run_mm(p2, wt[2], bt[2], a2, c2, relu=True)
    a3, c3 = _bn_affine(m3, v3, mlp[2]['g'], mlp[2]['be'])

    def att_w(name):
        p = att[name]
        return (jnp.transpose(p['w1']), p['b1'].reshape(1, -1), p['g1'],
                p['be1'], jnp.transpose(p['w2']), p['b2'].reshape(1, -1))

    wp1, bp1, gp1, ep1, wp2, bp2 = att_w('psi')
    wa1, ba1, ga1, ea1, wa2, ba2 = att_w('alpha')
    wf1, bf1, gf1, ef1, wf2, bf2 = att_w('fai')
    wg1, bg1, gg1, eg1, wg2, bg2 = att_w('gamma')

    cfa_args = (wt[0], bt[0], mlp[0]['g'].reshape(1, -1),
                mlp[0]['be'].reshape(1, -1),
                wt[1], bt[1], mlp[1]['g'].reshape(1, -1),
                mlp[1]['be'].reshape(1, -1),
                wt[2], bt[2], mlp[2]['g'].reshape(1, -1),
                mlp[2]['be'].reshape(1, -1),
                wf1, bf1, gf1.reshape(1, -1), ef1.reshape(1, -1),
                wf2, bf2)
    fai2 = _run_cfa(cg_all, cfa_args)

    pp1, mp1, vp1, pa1, ma1, va1 = _run_mm2(p3, a3, c3, wp1, bp1, wa1, ba1)
    ap1, cp1 = _bn_affine(mp1, vp1, gp1, ep1)
    aa1, ca1 = _bn_affine(ma1, va1, ga1, ea1)

    alpha2, pg1, mg1, vg1 = _run_pag(pp1, ap1, cp1, wp2, bp2,
                                     pa1, aa1, ca1, wa2, ba2,
                                     fai2, wg1, bg1)
    ag1, cg1 = _bn_affine(mg1, vg1, gg1, eg1)

    y = _run_final(pg1, ag1, cg1, wg2, bg2, alpha2)
    y = y.reshape(_B, _N_CENTER, 256)

    center_xyz = cc[:, :, 0:3]
    center_eula = cc[:, :, 3:6]
    center_near = cc[:, :, 6:8]
    center_meta = cc[:, :, 8:12]
    center_fea = cc[:, :, 12:76]
    new_fea_out = jnp.concatenate([center_fea, y], axis=2)
    return center_xyz, center_eula, center_near, center_meta, new_fea_out


# speedup vs baseline: 1.0826x; 1.0826x over previous
import functools

import jax
import jax.numpy as jnp
from jax.experimental import pallas as pl
from jax.experimental.pallas import tpu as pltpu
from jax.experimental.pallas import tpu_sc as plsc

_N_CENTER = 512
_N_NEAR = 32
_B = 8
_N = 2048
_SBLK = 16
_NSB = _N_CENTER // _SBLK
_BR = 1024
_ROWS = _B * _N_CENTER * _N_NEAR
_CROWS = _B * _N_CENTER
_F_CH = 76
_IN_CH = 82
_EPS = 1e-5


def _dot(a, b):
    return jax.lax.dot_general(a, b, (((1,), (0,)), ((), ())),
                               preferred_element_type=jnp.float32)


def _dotb(a, b):
    return jax.lax.dot_general(a.astype(jnp.bfloat16), b.astype(jnp.bfloat16),
                               (((1,), (0,)), ((), ())),
                               preferred_element_type=jnp.float32)


def _dot_t(a, b):
    return jax.lax.dot_general(a, b, (((1,), (1,)), ((), ())),
                               preferred_element_type=jnp.float32)


def _fps_body(xs_ref, ys_ref, zs_ref, f0_ref, out_ref):
    xs = xs_ref[...]
    ys = ys_ref[...]
    zs = zs_ref[...]
    col = jax.lax.broadcasted_iota(jnp.int32, (_B, _N), 1)
    lane_c = jax.lax.broadcasted_iota(jnp.int32, (_B, _N_CENTER), 1)

    def step(t, carry):
        dist, far, cents = carry
        cents = jnp.where(lane_c == t, far, cents)
        sel = col == far
        cx = jnp.sum(jnp.where(sel, xs, 0.0), axis=1, keepdims=True)
        cy = jnp.sum(jnp.where(sel, ys, 0.0), axis=1, keepdims=True)
        cz = jnp.sum(jnp.where(sel, zs, 0.0), axis=1, keepdims=True)
        dx = xs - cx
        dy = ys - cy
        dz = zs - cz
        d = (dx * dx + dy * dy) + dz * dz
        dist = jnp.minimum(dist, d)
        m = jnp.max(dist, axis=1, keepdims=True)
        far = jnp.min(jnp.where(dist == m, col, _N), axis=1, keepdims=True)
        return dist, far, cents

    dist0 = jnp.full((_B, _N), 1e10, jnp.float32)
    far0 = f0_ref[...]
    cents0 = jnp.zeros((_B, _N_CENTER), jnp.int32)
    _, _, cents = jax.lax.fori_loop(0, _N_CENTER, step, (dist0, far0, cents0))
    out_ref[...] = cents


def _run_fps(xs, ys, zs, f0):
    return pl.pallas_call(
        _fps_body,
        out_shape=jax.ShapeDtypeStruct((_B, _N_CENTER), jnp.int32),
    )(xs, ys, zs, f0)


def _knn_body(p_ref, xs_ref, ys_ref, zs_ref, fps_ref, out_ref):
    p = p_ref[0]
    xs = xs_ref[0]
    ys = ys_ref[0]
    zs = zs_ref[0]
    fpsc = fps_ref[0]

    colc = jax.lax.broadcasted_iota(jnp.int32, (_N_CENTER, _N), 1)
    oh = (colc == fpsc).astype(jnp.float32)
    px = p[:, 0:1]
    py = p[:, 1:2]
    pz = p[:, 2:3]
    sqf = (px * px + py * py) + pz * pz
    aug = jnp.concatenate([p, sqf], axis=1)
    cg = _dot(oh, aug)
    c = cg[:, 0:3]
    sqc = cg[:, 3:4]
    sq_row = (xs * xs + ys * ys) + zs * zs
    dots = _dot_t(c, p)
    d = (sqc + sq_row) - 2.0 * dots

    lane_k = jax.lax.broadcasted_iota(jnp.int32, (_N_CENTER, _N_NEAR), 1)

    def step(k, carry):
        d_cur, acc = carry
        m = jnp.min(d_cur, axis=1, keepdims=True)
        j = jnp.min(jnp.where(d_cur == m, colc, _N), axis=1, keepdims=True)
        acc = jnp.where(lane_k == k, j, acc)
        d_cur = jnp.where(colc == j, jnp.inf, d_cur)
        return d_cur, acc

    acc0 = jnp.zeros((_N_CENTER, _N_NEAR), jnp.int32)
    carry = (d, acc0)
    for k in range(_N_NEAR):
        carry = step(k, carry)
    out_ref[0] = carry[1]


def _run_knn(xyz, xs, ys, zs, fps3):
    return pl.pallas_call(
        _knn_body,
        grid=(_B,),
        in_specs=[
            pl.BlockSpec((1, _N, 3), lambda b: (b, 0, 0)),
            pl.BlockSpec((1, 1, _N), lambda b: (b, 0, 0)),
            pl.BlockSpec((1, 1, _N), lambda b: (b, 0, 0)),
            pl.BlockSpec((1, 1, _N), lambda b: (b, 0, 0)),
            pl.BlockSpec((1, _N_CENTER, 1), lambda b: (b, 0, 0)),
        ],
        out_specs=pl.BlockSpec((1, _N_CENTER, _N_NEAR), lambda b: (b, 0, 0)),
        out_shape=jax.ShapeDtypeStruct((_B, _N_CENTER, _N_NEAR), jnp.int32),
    )(xyz, xs, ys, zs, fps3)


_GROWS = _ROWS + _CROWS
_NW = 32
_RPW = _GROWS // _NW
_GCH = 6
_CHUNK = _RPW // _GCH
_PCH = 128


def _run_sc_gather(table, gidx):
    mesh = plsc.VectorSubcoreMesh(core_axis_name="c", subcore_axis_name="s")

    @functools.partial(
        pl.kernel, mesh=mesh,
        out_type=jax.ShapeDtypeStruct((_GROWS, _PCH), jnp.float32),
        scratch_types=[
            pltpu.VMEM((_CHUNK,), jnp.int32),
            pltpu.VMEM((_CHUNK, _PCH), jnp.float32),
            pltpu.SemaphoreType.DMA,
        ],
    )
    def k(table_hbm, idx_hbm, out_hbm, idx_v, rows_v, sem):
        wid = jax.lax.axis_index("s") * 2 + jax.lax.axis_index("c")
        base = wid * _RPW

        def body(i, carry):
            off = base + i * _CHUNK
            pltpu.sync_copy(idx_hbm.at[pl.ds(off, _CHUNK)], idx_v)
            pltpu.async_copy(table_hbm.at[idx_v], rows_v, sem).wait()
            pltpu.sync_copy(rows_v, out_hbm.at[pl.ds(off, _CHUNK)])
            return carry

        jax.lax.fori_loop(0, _GCH, body, 0)

    return k(table, gidx)


def _l1_body(xg_ref, cg_ref, w_ref, b_ref, o_ref, s1_ref, s2_ref):
    xg = xg_ref[0, 0]
    cg = cg_ref[0, 0]

    cb = jnp.broadcast_to(cg.reshape(_SBLK, 1, _PCH),
                          (_SBLK, _N_NEAR, _PCH)).reshape(
                              _SBLK * _N_NEAR, _PCH)
    x0 = jnp.concatenate([
        xg[:, 0:3] - cb[:, 0:3],
        xg[:, 3:6] - cb[:, 3:6],
        xg[:, 6:8], cb[:, 6:8],
        xg[:, 8:12], cb[:, 8:12],
        xg[:, 12:76],
    ], axis=1)
    p = _dotb(x0, w_ref[...]) + b_ref[...]
    o_ref[0, 0] = p.astype(jnp.bfloat16)
    s1_ref[0, 0] = jnp.sum(p, axis=0, keepdims=True)
    s2_ref[0, 0] = jnp.sum(p * p, axis=0, keepdims=True)


def _run_l1(xg4, cg4, w, b2):
    cout = w.shape[1]
    outs = pl.pallas_call(
        _l1_body,
        grid=(_B, _NSB),
        in_specs=[
            pl.BlockSpec((1, 1, _SBLK * _N_NEAR, _PCH),
                         lambda b, s: (b, s, 0, 0)),
            pl.BlockSpec((1, 1, _SBLK, _PCH), lambda b, s: (b, s, 0, 0)),
            pl.BlockSpec((_IN_CH, cout), lambda b, s: (0, 0)),
            pl.BlockSpec((1, cout), lambda b, s: (0, 0)),
        ],
        out_specs=[
            pl.BlockSpec((1, 1, _SBLK * _N_NEAR, cout),
                         lambda b, s: (b, s, 0, 0)),
            pl.BlockSpec((1, 1, 1, cout), lambda b, s: (b, s, 0, 0)),
            pl.BlockSpec((1, 1, 1, cout), lambda b, s: (b, s, 0, 0)),
        ],
        out_shape=[
            jax.ShapeDtypeStruct((_B, _NSB, _SBLK * _N_NEAR, cout),
                                 jnp.bfloat16),
            jax.ShapeDtypeStruct((_B, _NSB, 1, cout), jnp.float32),
            jax.ShapeDtypeStruct((_B, _NSB, 1, cout), jnp.float32),
        ],
    )(xg4, cg4, w, b2)
    p, s1, s2 = outs
    p = p.reshape(_ROWS, cout)
    mean = jnp.sum(s1, axis=(0, 1, 2)) / _ROWS
    var = jnp.sum(s2, axis=(0, 1, 2)) / _ROWS - mean * mean
    return p, mean, var


def _mm_body(aff, relu, stats, x_ref, a_ref, c_ref, w_ref, b_ref,
             o_ref, s1_ref, s2_ref):
    h = x_ref[...].astype(jnp.float32)
    if aff:
        h = h * a_ref[...] + c_ref[...]
    if relu:
        h = jnp.maximum(h, 0.0)
    p = _dotb(h, w_ref[...]) + b_ref[...]
    o_ref[...] = p.astype(jnp.bfloat16)
    if stats:
        s1_ref[0] = jnp.sum(p, axis=0, keepdims=True)
        s2_ref[0] = jnp.sum(p * p, axis=0, keepdims=True)


def _run_mm(x, w, b2, a2=None, c2=None, relu=False, stats=True):
    rows, cin = x.shape
    cout = w.shape[1]
    g = rows // _BR
    aff = a2 is not None
    if not aff:
        a2 = jnp.zeros((1, cin), jnp.float32)
        c2 = jnp.zeros((1, cin), jnp.float32)
    body = functools.partial(_mm_body, aff, relu, stats)
    outs = pl.pallas_call(
        body,
        grid=(g,),
        in_specs=[
            pl.BlockSpec((_BR, cin), lambda i: (i, 0)),
            pl.BlockSpec((1, cin), lambda i: (0, 0)),
            pl.BlockSpec((1, cin), lambda i: (0, 0)),
            pl.BlockSpec((cin, cout), lambda i: (0, 0)),
            pl.BlockSpec((1, cout), lambda i: (0, 0)),
        ],
        out_specs=[
            pl.BlockSpec((_BR, cout), lambda i: (i, 0)),
            pl.BlockSpec((1, 1, cout), lambda i: (i, 0, 0)),
            pl.BlockSpec((1, 1, cout), lambda i: (i, 0, 0)),
        ],
        out_shape=[
            jax.ShapeDtypeStruct((rows, cout), jnp.bfloat16),
            jax.ShapeDtypeStruct((g, 1, cout), jnp.float32),
            jax.ShapeDtypeStruct((g, 1, cout), jnp.float32),
        ],
    )(x, a2, c2, w, b2)
    p, s1, s2 = outs
    if not stats:
        return p, None, None
    mean = jnp.sum(s1, axis=(0, 1)) / rows
    var = jnp.sum(s2, axis=(0, 1)) / rows - mean * mean
    return p, mean, var


def _mm2_body(x_ref, a_ref, c_ref, w1_ref, b1_ref, w2_ref, b2_ref,
              o1_ref, o2_ref, s11_ref, s12_ref, s21_ref, s22_ref):
    h = jnp.maximum(x_ref[...].astype(jnp.float32) * a_ref[...] +
                    c_ref[...], 0.0)
    p1 = _dotb(h, w1_ref[...]) + b1_ref[...]
    p2 = _dotb(h, w2_ref[...]) + b2_ref[...]
    o1_ref[...] = p1.astype(jnp.bfloat16)
    o2_ref[...] = p2.astype(jnp.bfloat16)
    s11_ref[0] = jnp.sum(p1, axis=0, keepdims=True)
    s12_ref[0] = jnp.sum(p1 * p1, axis=0, keepdims=True)
    s21_ref[0] = jnp.sum(p2, axis=0, keepdims=True)
    s22_ref[0] = jnp.sum(p2 * p2, axis=0, keepdims=True)


def _run_mm2(x, a2, c2, w1, b1, w2, b2):
    rows, cin = x.shape
    cout = w1.shape[1]
    g = rows // _BR
    bcast = lambda i: (0, 0)
    outs = pl.pallas_call(
        _mm2_body,
        grid=(g,),
        in_specs=[
            pl.BlockSpec((_BR, cin), lambda i: (i, 0)),
            pl.BlockSpec((1, cin), bcast),
            pl.BlockSpec((1, cin), bcast),
            pl.BlockSpec((cin, cout), bcast),
            pl.BlockSpec((1, cout), bcast),
            pl.BlockSpec((cin, cout), bcast),
            pl.BlockSpec((1, cout), bcast),
        ],
        out_specs=[
            pl.BlockSpec((_BR, cout), lambda i: (i, 0)),
            pl.BlockSpec((_BR, cout), lambda i: (i, 0)),
            pl.BlockSpec((1, 1, cout), lambda i: (i, 0, 0)),
            pl.BlockSpec((1, 1, cout), lambda i: (i, 0, 0)),
            pl.BlockSpec((1, 1, cout), lambda i: (i, 0, 0)),
            pl.BlockSpec((1, 1, cout), lambda i: (i, 0, 0)),
        ],
        out_shape=[
            jax.ShapeDtypeStruct((rows, cout), jnp.bfloat16),
            jax.ShapeDtypeStruct((rows, cout), jnp.bfloat16),
            jax.ShapeDtypeStruct((g, 1, cout), jnp.float32),
            jax.ShapeDtypeStruct((g, 1, cout), jnp.float32),
            jax.ShapeDtypeStruct((g, 1, cout), jnp.float32),
            jax.ShapeDtypeStruct((g, 1, cout), jnp.float32),
        ],
    )(x, a2, c2, w1, b1, w2, b2)
    p1, p2, s11, s12, s21, s22 = outs
    m1 = jnp.sum(s11, axis=(0, 1)) / rows
    v1 = jnp.sum(s12, axis=(0, 1)) / rows - m1 * m1
    m2 = jnp.sum(s21, axis=(0, 1)) / rows
    v2 = jnp.sum(s22, axis=(0, 1)) / rows - m2 * m2
    return p1, m1, v1, p2, m2, v2


def _pag_body(pp_ref, ap_ref, cp_ref, wp2_ref, bp2_ref,
              pa_ref, aa_ref, ca_ref, wa2_ref, ba2_ref,
              fai_ref, wg1_ref, bg1_ref,
              al_ref, pg_ref, s1_ref, s2_ref):
    hp = jnp.maximum(pp_ref[...].astype(jnp.float32) * ap_ref[...] +
                     cp_ref[...], 0.0)
    psi2 = _dotb(hp, wp2_ref[...]) + bp2_ref[...]
    ha = jnp.maximum(pa_ref[...].astype(jnp.float32) * aa_ref[...] +
                     ca_ref[...], 0.0)
    al_ref[...] = (_dotb(ha, wa2_ref[...]) + ba2_ref[...]).astype(jnp.bfloat16)
    gc = _BR // _N_NEAR
    gin = (fai_ref[...].reshape(gc, 1, 256) -
           psi2.reshape(gc, _N_NEAR, 256)).reshape(_BR, 256)
    pg = _dotb(gin, wg1_ref[...]) + bg1_ref[...]
    pg_ref[...] = pg.astype(jnp.bfloat16)
    s1_ref[0] = jnp.sum(pg, axis=0, keepdims=True)
    s2_ref[0] = jnp.sum(pg * pg, axis=0, keepdims=True)


def _run_pag(pp1, ap, cp, wp2, bp2, pa1, aa, ca, wa2, ba2, fai2, wg1, bg1):
    g = _ROWS // _BR
    gc = _BR // _N_NEAR
    bcast2 = lambda i: (0, 0)
    outs = pl.pallas_call(
        _pag_body,
        grid=(g,),
        in_specs=[
            pl.BlockSpec((_BR, 264), lambda i: (i, 0)),
            pl.BlockSpec((1, 264), bcast2),
            pl.BlockSpec((1, 264), bcast2),
            pl.BlockSpec((264, 256), bcast2),
            pl.BlockSpec((1, 256), bcast2),
            pl.BlockSpec((_BR, 264), lambda i: (i, 0)),
            pl.BlockSpec((1, 264), bcast2),
            pl.BlockSpec((1, 264), bcast2),
            pl.BlockSpec((264, 256), bcast2),
            pl.BlockSpec((1, 256), bcast2),
            pl.BlockSpec((gc, 256), lambda i: (i, 0)),
            pl.BlockSpec((256, 264), bcast2),
            pl.BlockSpec((1, 264), bcast2),
        ],
        out_specs=[
            pl.BlockSpec((_BR, 256), lambda i: (i, 0)),
            pl.BlockSpec((_BR, 264), lambda i: (i, 0)),
            pl.BlockSpec((1, 1, 264), lambda i: (i, 0, 0)),
            pl.BlockSpec((1, 1, 264), lambda i: (i, 0, 0)),
        ],
        out_shape=[
            jax.ShapeDtypeStruct((_ROWS, 256), jnp.bfloat16),
            jax.ShapeDtypeStruct((_ROWS, 264), jnp.bfloat16),
            jax.ShapeDtypeStruct((g, 1, 264), jnp.float32),
            jax.ShapeDtypeStruct((g, 1, 264), jnp.float32),
        ],
    )(pp1, ap, cp, wp2, bp2, pa1, aa, ca, wa2, ba2, fai2, wg1, bg1)
    alpha2, pg1, s1, s2 = outs
    mean = jnp.sum(s1, axis=(0, 1)) / _ROWS
    var = jnp.sum(s2, axis=(0, 1)) / _ROWS - mean * mean
    return alpha2, pg1, mean, var


def _bn_affine(mean, var, g, be):
    a = g / jnp.sqrt(var + _EPS)
    c = be - mean * a
    return a.reshape(1, -1), c.reshape(1, -1)


def _cfa_body(cg_ref, w1_ref, b1_ref, g1_ref, e1_ref, w2_ref, b2_ref,
              g2_ref, e2_ref, w3_ref, b3_ref, g3_ref, e3_ref,
              wf1_ref, bf1_ref, gf1_ref, ef1_ref, wf2_ref, bf2_ref,
              out_ref):
    cg = cg_ref[...]
    x = jnp.concatenate([
        jnp.zeros((_CROWS, 6), jnp.float32),
        cg[:, 6:8], cg[:, 6:8],
        cg[:, 8:12], cg[:, 8:12],
        cg[:, 12:76],
    ], axis=1)

    def bnrelu(p, g, e):
        m = jnp.mean(p, axis=0, keepdims=True)
        v = jnp.mean((p - m) * (p - m), axis=0, keepdims=True)
        return jnp.maximum(g * (p - m) / jnp.sqrt(v + _EPS) + e, 0.0)

    x = bnrelu(_dotb(x, w1_ref[...]) + b1_ref[...], g1_ref[...], e1_ref[...])
    x = bnrelu(_dotb(x, w2_ref[...]) + b2_ref[...], g2_ref[...], e2_ref[...])
    x = bnrelu(_dotb(x, w3_ref[...]) + b3_ref[...], g3_ref[...], e3_ref[...])
    x = bnrelu(_dotb(x, wf1_ref[...]) + bf1_ref[...], gf1_ref[...],
               ef1_ref[...])
    out_ref[...] = _dotb(x, wf2_ref[...]) + bf2_ref[...]


def _run_cfa(cg_all, args):
    return pl.pallas_call(
        _cfa_body,
        out_shape=jax.ShapeDtypeStruct((_CROWS, 256), jnp.float32),
    )(cg_all, *args)


def _final_body(pg_ref, a_ref, c_ref, w_ref, b_ref, al_ref, y_ref):
    h = jnp.maximum(pg_ref[...].astype(jnp.float32) * a_ref[...] +
                    c_ref[...], 0.0)
    gam = _dotb(h, w_ref[...]) + b_ref[...]
    m = jnp.max(gam, axis=1, keepdims=True)
    e = jnp.exp(gam - m)
    s = jnp.sum(e, axis=1, keepdims=True)
    contrib = (e / s) * al_ref[...].astype(jnp.float32) * (256.0 / _N_NEAR)
    gc = _BR // _N_NEAR
    y_ref[...] = jnp.sum(contrib.reshape(gc, _N_NEAR, 256), axis=1)


def _run_final(pg1, ag, cg, w, b2, alpha2):
    g = _ROWS // _BR
    gc = _BR // _N_NEAR
    return pl.pallas_call(
        _final_body,
        grid=(g,),
        in_specs=[
            pl.BlockSpec((_BR, 264), lambda i: (i, 0)),
            pl.BlockSpec((1, 264), lambda i: (0, 0)),
            pl.BlockSpec((1, 264), lambda i: (0, 0)),
            pl.BlockSpec((264, 256), lambda i: (0, 0)),
            pl.BlockSpec((1, 256), lambda i: (0, 0)),
            pl.BlockSpec((_BR, 256), lambda i: (i, 0)),
        ],
        out_specs=pl.BlockSpec((gc, 256), lambda i: (i, 0)),
        out_shape=jax.ShapeDtypeStruct((_CROWS, 256), jnp.float32),
    )(pg1, ag, cg, w, b2, alpha2)


@jax.jit
def kernel(xyz, eula_angle, edge_nearby, meta_type, fea, params):
    xs = xyz[:, :, 0]
    ys = xyz[:, :, 1]
    zs = xyz[:, :, 2]
    f0 = jax.random.randint(jax.random.key(42), (_B,), 0, _N,
                            dtype=jnp.int32).reshape(_B, 1)

    fps_idx = _run_fps(xs, ys, zs, f0)
    fps3 = fps_idx.reshape(_B, _N_CENTER, 1)
    xs3 = xs.reshape(_B, 1, _N)
    ys3 = ys.reshape(_B, 1, _N)
    zs3 = zs.reshape(_B, 1, _N)
    idx = _run_knn(xyz, xs3, ys3, zs3, fps3)

    feat = jnp.concatenate([xyz, eula_angle, edge_nearby, meta_type, fea,
                            jnp.zeros((_B, _N, _PCH - _F_CH), jnp.float32)],
                           axis=2)
    boff = (jnp.arange(_B, dtype=jnp.int32) * _N).reshape(_B, 1, 1)
    gidx = jnp.concatenate([
        (idx + boff).reshape(-1),
        (fps_idx + boff[:, :, 0]).reshape(-1),
    ])
    gath = _run_sc_gather(feat.reshape(_B * _N, _PCH), gidx)
    xg4 = gath[:_ROWS].reshape(_B, _NSB, _SBLK * _N_NEAR, _PCH)
    cg_all = gath[_ROWS:]
    cg4 = cg_all.reshape(_B, _NSB, _SBLK, _PCH)
    cc = cg_all.reshape(_B, _N_CENTER, _PCH)

    mlp = params['mlp']
    att = params['att']
    wt = [jnp.transpose(layer['w']) for layer in mlp]
    bt = [layer['b'].reshape(1, -1) for layer in mlp]

    p1, m1, v1 = _run_l1(xg4, cg4, wt[0], bt[0])
    a1, c1 = _bn_affine(m1, v1, mlp[0]['g'], mlp[0]['be'])
    p2, m2, v2 = _run_mm(p1, wt[1], bt[1], a1, c1, relu=True)
    a2, c2 = _bn_affine(m2, v2, mlp[1]['g'], mlp[1]['be'])
    p3, m3, v3 = _run_mm(p2, wt[2], bt[2], a2, c2, relu=True)
    a3, c3 = _bn_affine(m3, v3, mlp[2]['g'], mlp[2]['be'])

    def att_w(name):
        p = att[name]
        return (jnp.transpose(p['w1']), p['b1'].reshape(1, -1), p['g1'],
                p['be1'], jnp.transpose(p['w2']), p['b2'].reshape(1, -1))

    wp1, bp1, gp1, ep1, wp2, bp2 = att_w('psi')
    wa1, ba1, ga1, ea1, wa2, ba2 = att_w('alpha')
    wf1, bf1, gf1, ef1, wf2, bf2 = att_w('fai')
    wg1, bg1, gg1, eg1, wg2, bg2 = att_w('gamma')

    cfa_args = (wt[0], bt[0], mlp[0]['g'].reshape(1, -1),
                mlp[0]['be'].reshape(1, -1),
                wt[1], bt[1], mlp[1]['g'].reshape(1, -1),
                mlp[1]['be'].reshape(1, -1),
                wt[2], bt[2], mlp[2]['g'].reshape(1, -1),
                mlp[2]['be'].reshape(1, -1),
                wf1, bf1, gf1.reshape(1, -1), ef1.reshape(1, -1),
                wf2, bf2)
    fai2 = _run_cfa(cg_all, cfa_args)

    pp1, mp1, vp1, pa1, ma1, va1 = _run_mm2(p3, a3, c3, wp1, bp1, wa1, ba1)
    ap1, cp1 = _bn_affine(mp1, vp1, gp1, ep1)
    aa1, ca1 = _bn_affine(ma1, va1, ga1, ea1)

    alpha2, pg1, mg1, vg1 = _run_pag(pp1, ap1, cp1, wp2, bp2,
                                     pa1, aa1, ca1, wa2, ba2,
                                     fai2, wg1, bg1)
    ag1, cg1 = _bn_affine(mg1, vg1, gg1, eg1)

    y = _run_final(pg1, ag1, cg1, wg2, bg2, alpha2)
    y = y.reshape(_B, _N_CENTER, 256)

    center_xyz = cc[:, :, 0:3]
    center_eula = cc[:, :, 3:6]
    center_near = cc[:, :, 6:8]
    center_meta = cc[:, :, 8:12]
    center_fea = cc[:, :, 12:76]
    new_fea_out = jnp.concatenate([center_fea, y], axis=2)
    return center_xyz, center_eula, center_near, center_meta, new_fea_out


# speedup vs baseline: 1.1015x; 1.0174x over previous
import functools

import jax
import jax.numpy as jnp
from jax.experimental import pallas as pl
from jax.experimental.pallas import tpu as pltpu
from jax.experimental.pallas import tpu_sc as plsc

_N_CENTER = 512
_N_NEAR = 32
_B = 8
_N = 2048
_SBLK = 16
_NSB = _N_CENTER // _SBLK
_BR = 1024
_ROWS = _B * _N_CENTER * _N_NEAR
_CROWS = _B * _N_CENTER
_F_CH = 76
_IN_CH = 82
_EPS = 1e-5


def _dot(a, b):
    return jax.lax.dot_general(a, b, (((1,), (0,)), ((), ())),
                               preferred_element_type=jnp.float32)


def _dotb(a, b):
    return jax.lax.dot_general(a.astype(jnp.bfloat16), b.astype(jnp.bfloat16),
                               (((1,), (0,)), ((), ())),
                               preferred_element_type=jnp.float32)


def _dot_t(a, b):
    return jax.lax.dot_general(a, b, (((1,), (1,)), ((), ())),
                               preferred_element_type=jnp.float32)


def _fps_body(xs_ref, ys_ref, zs_ref, f0_ref, out_ref):
    xs = xs_ref[...]
    ys = ys_ref[...]
    zs = zs_ref[...]
    col = jax.lax.broadcasted_iota(jnp.int32, (_B, _N), 1)
    lane_c = jax.lax.broadcasted_iota(jnp.int32, (_B, _N_CENTER), 1)

    def step(t, carry):
        dist, far, cents = carry
        cents = jnp.where(lane_c == t, far, cents)
        sel = col == far
        cx = jnp.sum(jnp.where(sel, xs, 0.0), axis=1, keepdims=True)
        cy = jnp.sum(jnp.where(sel, ys, 0.0), axis=1, keepdims=True)
        cz = jnp.sum(jnp.where(sel, zs, 0.0), axis=1, keepdims=True)
        dx = xs - cx
        dy = ys - cy
        dz = zs - cz
        d = (dx * dx + dy * dy) + dz * dz
        dist = jnp.minimum(dist, d)
        m = jnp.max(dist, axis=1, keepdims=True)
        far = jnp.min(jnp.where(dist == m, col, _N), axis=1, keepdims=True)
        return dist, far, cents

    dist0 = jnp.full((_B, _N), 1e10, jnp.float32)
    far0 = f0_ref[...]
    cents0 = jnp.zeros((_B, _N_CENTER), jnp.int32)

    def step4(q, carry):
        t0 = q * 4
        for u in range(4):
            carry = step(t0 + u, carry)
        return carry

    _, _, cents = jax.lax.fori_loop(0, _N_CENTER // 4, step4,
                                    (dist0, far0, cents0))
    out_ref[...] = cents


def _run_fps(xs, ys, zs, f0):
    return pl.pallas_call(
        _fps_body,
        out_shape=jax.ShapeDtypeStruct((_B, _N_CENTER), jnp.int32),
    )(xs, ys, zs, f0)


def _knn_body(p_ref, xs_ref, ys_ref, zs_ref, fps_ref, out_ref):
    p = p_ref[0]
    xs = xs_ref[0]
    ys = ys_ref[0]
    zs = zs_ref[0]
    fpsc = fps_ref[0]

    colc = jax.lax.broadcasted_iota(jnp.int32, (_N_CENTER, _N), 1)
    oh = (colc == fpsc).astype(jnp.float32)
    px = p[:, 0:1]
    py = p[:, 1:2]
    pz = p[:, 2:3]
    sqf = (px * px + py * py) + pz * pz
    aug = jnp.concatenate([p, sqf], axis=1)
    cg = _dot(oh, aug)
    c = cg[:, 0:3]
    sqc = cg[:, 3:4]
    sq_row = (xs * xs + ys * ys) + zs * zs
    dots = _dot_t(c, p)
    d = (sqc + sq_row) - 2.0 * dots

    lane_k = jax.lax.broadcasted_iota(jnp.int32, (_N_CENTER, _N_NEAR), 1)

    def step(k, carry):
        d_cur, acc = carry
        m = jnp.min(d_cur, axis=1, keepdims=True)
        j = jnp.min(jnp.where(d_cur == m, colc, _N), axis=1, keepdims=True)
        acc = jnp.where(lane_k == k, j, acc)
        d_cur = jnp.where(colc == j, jnp.inf, d_cur)
        return d_cur, acc

    acc0 = jnp.zeros((_N_CENTER, _N_NEAR), jnp.int32)
    carry = (d, acc0)
    for k in range(_N_NEAR):
        carry = step(k, carry)
    out_ref[0] = carry[1]


def _run_knn(xyz, xs, ys, zs, fps3):
    return pl.pallas_call(
        _knn_body,
        grid=(_B,),
        in_specs=[
            pl.BlockSpec((1, _N, 3), lambda b: (b, 0, 0)),
            pl.BlockSpec((1, 1, _N), lambda b: (b, 0, 0)),
            pl.BlockSpec((1, 1, _N), lambda b: (b, 0, 0)),
            pl.BlockSpec((1, 1, _N), lambda b: (b, 0, 0)),
            pl.BlockSpec((1, _N_CENTER, 1), lambda b: (b, 0, 0)),
        ],
        out_specs=pl.BlockSpec((1, _N_CENTER, _N_NEAR), lambda b: (b, 0, 0)),
        out_shape=jax.ShapeDtypeStruct((_B, _N_CENTER, _N_NEAR), jnp.int32),
    )(xyz, xs, ys, zs, fps3)


_GROWS = _ROWS + _CROWS
_NW = 32
_RPW = _GROWS // _NW
_GCH = 12
_CHUNK = _RPW // _GCH
_PCH = 128


def _run_sc_gather(table, gidx):
    mesh = plsc.VectorSubcoreMesh(core_axis_name="c", subcore_axis_name="s")

    @functools.partial(
        pl.kernel, mesh=mesh,
        out_type=jax.ShapeDtypeStruct((_GROWS, _PCH), jnp.float32),
        scratch_types=[
            pltpu.VMEM((_RPW,), jnp.int32),
            pltpu.VMEM((_CHUNK, _PCH), jnp.float32),
            pltpu.VMEM((_CHUNK, _PCH), jnp.float32),
            pltpu.SemaphoreType.DMA,
            pltpu.SemaphoreType.DMA,
        ],
    )
    def k(table_hbm, idx_hbm, out_hbm, idx_v, buf0, buf1, sem0, sem1):
        wid = jax.lax.axis_index("s") * 2 + jax.lax.axis_index("c")
        base = wid * _RPW
        pltpu.sync_copy(idx_hbm.at[pl.ds(base, _RPW)], idx_v)
        bufs = (buf0, buf1)
        sems = (sem0, sem1)
        cps = [None, None]
        cps[0] = pltpu.async_copy(
            table_hbm.at[idx_v.at[pl.ds(0, _CHUNK)]], bufs[0], sems[0])
        for i in range(_GCH):
            cps[i % 2].wait()
            if i + 1 < _GCH:
                cps[(i + 1) % 2] = pltpu.async_copy(
                    table_hbm.at[idx_v.at[pl.ds((i + 1) * _CHUNK, _CHUNK)]],
                    bufs[(i + 1) % 2], sems[(i + 1) % 2])
            pltpu.sync_copy(bufs[i % 2],
                            out_hbm.at[pl.ds(base + i * _CHUNK, _CHUNK)])

    return k(table, gidx)


def _l1_body(xg_ref, cg_ref, w_ref, b_ref, o_ref, s1_ref, s2_ref):
    xg = xg_ref[0, 0]
    cg = cg_ref[0, 0]

    cb = jnp.broadcast_to(cg.reshape(_SBLK, 1, _PCH),
                          (_SBLK, _N_NEAR, _PCH)).reshape(
                              _SBLK * _N_NEAR, _PCH)
    x0 = jnp.concatenate([
        xg[:, 0:3] - cb[:, 0:3],
        xg[:, 3:6] - cb[:, 3:6],
        xg[:, 6:8], cb[:, 6:8],
        xg[:, 8:12], cb[:, 8:12],
        xg[:, 12:76],
    ], axis=1)
    p = _dotb(x0, w_ref[...]) + b_ref[...]
    o_ref[0, 0] = p.astype(jnp.bfloat16)
    s1_ref[0, 0] = jnp.sum(p, axis=0, keepdims=True)
    s2_ref[0, 0] = jnp.sum(p * p, axis=0, keepdims=True)


def _run_l1(xg4, cg4, w, b2):
    cout = w.shape[1]
    outs = pl.pallas_call(
        _l1_body,
        grid=(_B, _NSB),
        in_specs=[
            pl.BlockSpec((1, 1, _SBLK * _N_NEAR, _PCH),
                         lambda b, s: (b, s, 0, 0)),
            pl.BlockSpec((1, 1, _SBLK, _PCH), lambda b, s: (b, s, 0, 0)),
            pl.BlockSpec((_IN_CH, cout), lambda b, s: (0, 0)),
            pl.BlockSpec((1, cout), lambda b, s: (0, 0)),
        ],
        out_specs=[
            pl.BlockSpec((1, 1, _SBLK * _N_NEAR, cout),
                         lambda b, s: (b, s, 0, 0)),
            pl.BlockSpec((1, 1, 1, cout), lambda b, s: (b, s, 0, 0)),
            pl.BlockSpec((1, 1, 1, cout), lambda b, s: (b, s, 0, 0)),
        ],
        out_shape=[
            jax.ShapeDtypeStruct((_B, _NSB, _SBLK * _N_NEAR, cout),
                                 jnp.bfloat16),
            jax.ShapeDtypeStruct((_B, _NSB, 1, cout), jnp.float32),
            jax.ShapeDtypeStruct((_B, _NSB, 1, cout), jnp.float32),
        ],
    )(xg4, cg4, w, b2)
    p, s1, s2 = outs
    p = p.reshape(_ROWS, cout)
    mean = jnp.sum(s1, axis=(0, 1, 2)) / _ROWS
    var = jnp.sum(s2, axis=(0, 1, 2)) / _ROWS - mean * mean
    return p, mean, var


def _mm_body(aff, relu, stats, x_ref, a_ref, c_ref, w_ref, b_ref,
             o_ref, s1_ref, s2_ref):
    h = x_ref[...].astype(jnp.float32)
    if aff:
        h = h * a_ref[...] + c_ref[...]
    if relu:
        h = jnp.maximum(h, 0.0)
    p = _dotb(h, w_ref[...]) + b_ref[...]
    o_ref[...] = p.astype(jnp.bfloat16)
    if stats:
        s1_ref[0] = jnp.sum(p, axis=0, keepdims=True)
        s2_ref[0] = jnp.sum(p * p, axis=0, keepdims=True)


def _run_mm(x, w, b2, a2=None, c2=None, relu=False, stats=True):
    rows, cin = x.shape
    cout = w.shape[1]
    g = rows // _BR
    aff = a2 is not None
    if not aff:
        a2 = jnp.zeros((1, cin), jnp.float32)
        c2 = jnp.zeros((1, cin), jnp.float32)
    body = functools.partial(_mm_body, aff, relu, stats)
    outs = pl.pallas_call(
        body,
        grid=(g,),
        in_specs=[
            pl.BlockSpec((_BR, cin), lambda i: (i, 0)),
            pl.BlockSpec((1, cin), lambda i: (0, 0)),
            pl.BlockSpec((1, cin), lambda i: (0, 0)),
            pl.BlockSpec((cin, cout), lambda i: (0, 0)),
            pl.BlockSpec((1, cout), lambda i: (0, 0)),
        ],
        out_specs=[
            pl.BlockSpec((_BR, cout), lambda i: (i, 0)),
            pl.BlockSpec((1, 1, cout), lambda i: (i, 0, 0)),
            pl.BlockSpec((1, 1, cout), lambda i: (i, 0, 0)),
        ],
        out_shape=[
            jax.ShapeDtypeStruct((rows, cout), jnp.bfloat16),
            jax.ShapeDtypeStruct((g, 1, cout), jnp.float32),
            jax.ShapeDtypeStruct((g, 1, cout), jnp.float32),
        ],
    )(x, a2, c2, w, b2)
    p, s1, s2 = outs
    if not stats:
        return p, None, None
    mean = jnp.sum(s1, axis=(0, 1)) / rows
    var = jnp.sum(s2, axis=(0, 1)) / rows - mean * mean
    return p, mean, var


def _mm2_body(x_ref, a_ref, c_ref, w1_ref, b1_ref, w2_ref, b2_ref,
              o1_ref, o2_ref, s11_ref, s12_ref, s21_ref, s22_ref):
    h = jnp.maximum(x_ref[...].astype(jnp.float32) * a_ref[...] +
                    c_ref[...], 0.0)
    p1 = _dotb(h, w1_ref[...]) + b1_ref[...]
    p2 = _dotb(h, w2_ref[...]) + b2_ref[...]
    o1_ref[...] = p1.astype(jnp.bfloat16)
    o2_ref[...] = p2.astype(jnp.bfloat16)
    s11_ref[0] = jnp.sum(p1, axis=0, keepdims=True)
    s12_ref[0] = jnp.sum(p1 * p1, axis=0, keepdims=True)
    s21_ref[0] = jnp.sum(p2, axis=0, keepdims=True)
    s22_ref[0] = jnp.sum(p2 * p2, axis=0, keepdims=True)


def _run_mm2(x, a2, c2, w1, b1, w2, b2):
    rows, cin = x.shape
    cout = w1.shape[1]
    g = rows // _BR
    bcast = lambda i: (0, 0)
    outs = pl.pallas_call(
        _mm2_body,
        grid=(g,),
        in_specs=[
            pl.BlockSpec((_BR, cin), lambda i: (i, 0)),
            pl.BlockSpec((1, cin), bcast),
            pl.BlockSpec((1, cin), bcast),
            pl.BlockSpec((cin, cout), bcast),
            pl.BlockSpec((1, cout), bcast),
            pl.BlockSpec((cin, cout), bcast),
            pl.BlockSpec((1, cout), bcast),
        ],
        out_specs=[
            pl.BlockSpec((_BR, cout), lambda i: (i, 0)),
            pl.BlockSpec((_BR, cout), lambda i: (i, 0)),
            pl.BlockSpec((1, 1, cout), lambda i: (i, 0, 0)),
            pl.BlockSpec((1, 1, cout), lambda i: (i, 0, 0)),
            pl.BlockSpec((1, 1, cout), lambda i: (i, 0, 0)),
            pl.BlockSpec((1, 1, cout), lambda i: (i, 0, 0)),
        ],
        out_shape=[
            jax.ShapeDtypeStruct((rows, cout), jnp.bfloat16),
            jax.ShapeDtypeStruct((rows, cout), jnp.bfloat16),
            jax.ShapeDtypeStruct((g, 1, cout), jnp.float32),
            jax.ShapeDtypeStruct((g, 1, cout), jnp.float32),
            jax.ShapeDtypeStruct((g, 1, cout), jnp.float32),
            jax.ShapeDtypeStruct((g, 1, cout), jnp.float32),
        ],
    )(x, a2, c2, w1, b1, w2, b2)
    p1, p2, s11, s12, s21, s22 = outs
    m1 = jnp.sum(s11, axis=(0, 1)) / rows
    v1 = jnp.sum(s12, axis=(0, 1)) / rows - m1 * m1
    m2 = jnp.sum(s21, axis=(0, 1)) / rows
    v2 = jnp.sum(s22, axis=(0, 1)) / rows - m2 * m2
    return p1, m1, v1, p2, m2, v2


def _pag_body(pp_ref, ap_ref, cp_ref, wp2_ref, bp2_ref,
              pa_ref, aa_ref, ca_ref, wa2_ref, ba2_ref,
              fai_ref, wg1_ref, bg1_ref,
              al_ref, pg_ref, s1_ref, s2_ref):
    hp = jnp.maximum(pp_ref[...].astype(jnp.float32) * ap_ref[...] +
                     cp_ref[...], 0.0)
    psi2 = _dotb(hp, wp2_ref[...]) + bp2_ref[...]
    ha = jnp.maximum(pa_ref[...].astype(jnp.float32) * aa_ref[...] +
                     ca_ref[...], 0.0)
    al_ref[...] = (_dotb(ha, wa2_ref[...]) + ba2_ref[...]).astype(jnp.bfloat16)
    gc = _BR // _N_NEAR
    gin = (fai_ref[...].reshape(gc, 1, 256) -
           psi2.reshape(gc, _N_NEAR, 256)).reshape(_BR, 256)
    pg = _dotb(gin, wg1_ref[...]) + bg1_ref[...]
    pg_ref[...] = pg.astype(jnp.bfloat16)
    s1_ref[0] = jnp.sum(pg, axis=0, keepdims=True)
    s2_ref[0] = jnp.sum(pg * pg, axis=0, keepdims=True)


def _run_pag(pp1, ap, cp, wp2, bp2, pa1, aa, ca, wa2, ba2, fai2, wg1, bg1):
    g = _ROWS // _BR
    gc = _BR // _N_NEAR
    bcast2 = lambda i: (0, 0)
    outs = pl.pallas_call(
        _pag_body,
        grid=(g,),
        in_specs=[
            pl.BlockSpec((_BR, 264), lambda i: (i, 0)),
            pl.BlockSpec((1, 264), bcast2),
            pl.BlockSpec((1, 264), bcast2),
            pl.BlockSpec((264, 256), bcast2),
            pl.BlockSpec((1, 256), bcast2),
            pl.BlockSpec((_BR, 264), lambda i: (i, 0)),
            pl.BlockSpec((1, 264), bcast2),
            pl.BlockSpec((1, 264), bcast2),
            pl.BlockSpec((264, 256), bcast2),
            pl.BlockSpec((1, 256), bcast2),
            pl.BlockSpec((gc, 256), lambda i: (i, 0)),
            pl.BlockSpec((256, 264), bcast2),
            pl.BlockSpec((1, 264), bcast2),
        ],
        out_specs=[
            pl.BlockSpec((_BR, 256), lambda i: (i, 0)),
            pl.BlockSpec((_BR, 264), lambda i: (i, 0)),
            pl.BlockSpec((1, 1, 264), lambda i: (i, 0, 0)),
            pl.BlockSpec((1, 1, 264), lambda i: (i, 0, 0)),
        ],
        out_shape=[
            jax.ShapeDtypeStruct((_ROWS, 256), jnp.bfloat16),
            jax.ShapeDtypeStruct((_ROWS, 264), jnp.bfloat16),
            jax.ShapeDtypeStruct((g, 1, 264), jnp.float32),
            jax.ShapeDtypeStruct((g, 1, 264), jnp.float32),
        ],
    )(pp1, ap, cp, wp2, bp2, pa1, aa, ca, wa2, ba2, fai2, wg1, bg1)
    alpha2, pg1, s1, s2 = outs
    mean = jnp.sum(s1, axis=(0, 1)) / _ROWS
    var = jnp.sum(s2, axis=(0, 1)) / _ROWS - mean * mean
    return alpha2, pg1, mean, var


def _bn_affine(mean, var, g, be):
    a = g / jnp.sqrt(var + _EPS)
    c = be - mean * a
    return a.reshape(1, -1), c.reshape(1, -1)


def _cfa_body(cg_ref, w1_ref, b1_ref, g1_ref, e1_ref, w2_ref, b2_ref,
              g2_ref, e2_ref, w3_ref, b3_ref, g3_ref, e3_ref,
              wf1_ref, bf1_ref, gf1_ref, ef1_ref, wf2_ref, bf2_ref,
              out_ref):
    cg = cg_ref[...]
    x = jnp.concatenate([
        jnp.zeros((_CROWS, 6), jnp.float32),
        cg[:, 6:8], cg[:, 6:8],
        cg[:, 8:12], cg[:, 8:12],
        cg[:, 12:76],
    ], axis=1)

    def bnrelu(p, g, e):
        m = jnp.mean(p, axis=0, keepdims=True)
        v = jnp.mean((p - m) * (p - m), axis=0, keepdims=True)
        return jnp.maximum(g * (p - m) / jnp.sqrt(v + _EPS) + e, 0.0)

    x = bnrelu(_dotb(x, w1_ref[...]) + b1_ref[...], g1_ref[...], e1_ref[...])
    x = bnrelu(_dotb(x, w2_ref[...]) + b2_ref[...], g2_ref[...], e2_ref[...])
    x = bnrelu(_dotb(x, w3_ref[...]) + b3_ref[...], g3_ref[...], e3_ref[...])
    x = bnrelu(_dotb(x, wf1_ref[...]) + bf1_ref[...], gf1_ref[...],
               ef1_ref[...])
    out_ref[...] = _dotb(x, wf2_ref[...]) + bf2_ref[...]


def _run_cfa(cg_all, args):
    return pl.pallas_call(
        _cfa_body,
        out_shape=jax.ShapeDtypeStruct((_CROWS, 256), jnp.float32),
    )(cg_all, *args)


def _final_body(pg_ref, a_ref, c_ref, w_ref, b_ref, al_ref, y_ref):
    h = jnp.maximum(pg_ref[...].astype(jnp.float32) * a_ref[...] +
                    c_ref[...], 0.0)
    gam = _dotb(h, w_ref[...]) + b_ref[...]
    m = jnp.max(gam, axis=1, keepdims=True)
    e = jnp.exp(gam - m)
    s = jnp.sum(e, axis=1, keepdims=True)
    contrib = (e / s) * al_ref[...].astype(jnp.float32) * (256.0 / _N_NEAR)
    gc = _BR // _N_NEAR
    y_ref[...] = jnp.sum(contrib.reshape(gc, _N_NEAR, 256), axis=1)


def _run_final(pg1, ag, cg, w, b2, alpha2):
    g = _ROWS // _BR
    gc = _BR // _N_NEAR
    return pl.pallas_call(
        _final_body,
        grid=(g,),
        in_specs=[
            pl.BlockSpec((_BR, 264), lambda i: (i, 0)),
            pl.BlockSpec((1, 264), lambda i: (0, 0)),
            pl.BlockSpec((1, 264), lambda i: (0, 0)),
            pl.BlockSpec((264, 256), lambda i: (0, 0)),
            pl.BlockSpec((1, 256), lambda i: (0, 0)),
            pl.BlockSpec((_BR, 256), lambda i: (i, 0)),
        ],
        out_specs=pl.BlockSpec((gc, 256), lambda i: (i, 0)),
        out_shape=jax.ShapeDtypeStruct((_CROWS, 256), jnp.float32),
    )(pg1, ag, cg, w, b2, alpha2)


@jax.jit
def kernel(xyz, eula_angle, edge_nearby, meta_type, fea, params):
    xs = xyz[:, :, 0]
    ys = xyz[:, :, 1]
    zs = xyz[:, :, 2]
    f0 = jax.random.randint(jax.random.key(42), (_B,), 0, _N,
                            dtype=jnp.int32).reshape(_B, 1)

    fps_idx = _run_fps(xs, ys, zs, f0)
    fps3 = fps_idx.reshape(_B, _N_CENTER, 1)
    xs3 = xs.reshape(_B, 1, _N)
    ys3 = ys.reshape(_B, 1, _N)
    zs3 = zs.reshape(_B, 1, _N)
    idx = _run_knn(xyz, xs3, ys3, zs3, fps3)

    feat = jnp.concatenate([xyz, eula_angle, edge_nearby, meta_type, fea,
                            jnp.zeros((_B, _N, _PCH - _F_CH), jnp.float32)],
                           axis=2)
    boff = (jnp.arange(_B, dtype=jnp.int32) * _N).reshape(_B, 1, 1)
    gidx = jnp.concatenate([
        (idx + boff).reshape(-1),
        (fps_idx + boff[:, :, 0]).reshape(-1),
    ])
    gath = _run_sc_gather(feat.reshape(_B * _N, _PCH), gidx)
    xg4 = gath[:_ROWS].reshape(_B, _NSB, _SBLK * _N_NEAR, _PCH)
    cg_all = gath[_ROWS:]
    cg4 = cg_all.reshape(_B, _NSB, _SBLK, _PCH)
    cc = cg_all.reshape(_B, _N_CENTER, _PCH)

    mlp = params['mlp']
    att = params['att']
    wt = [jnp.transpose(layer['w']) for layer in mlp]
    bt = [layer['b'].reshape(1, -1) for layer in mlp]

    p1, m1, v1 = _run_l1(xg4, cg4, wt[0], bt[0])
    a1, c1 = _bn_affine(m1, v1, mlp[0]['g'], mlp[0]['be'])
    p2, m2, v2 = _run_mm(p1, wt[1], bt[1], a1, c1, relu=True)
    a2, c2 = _bn_affine(m2, v2, mlp[1]['g'], mlp[1]['be'])
    p3, m3, v3 = _run_mm(p2, wt[2], bt[2], a2, c2, relu=True)
    a3, c3 = _bn_affine(m3, v3, mlp[2]['g'], mlp[2]['be'])

    def att_w(name):
        p = att[name]
        return (jnp.transpose(p['w1']), p['b1'].reshape(1, -1), p['g1'],
                p['be1'], jnp.transpose(p['w2']), p['b2'].reshape(1, -1))

    wp1, bp1, gp1, ep1, wp2, bp2 = att_w('psi')
    wa1, ba1, ga1, ea1, wa2, ba2 = att_w('alpha')
    wf1, bf1, gf1, ef1, wf2, bf2 = att_w('fai')
    wg1, bg1, gg1, eg1, wg2, bg2 = att_w('gamma')

    cfa_args = (wt[0], bt[0], mlp[0]['g'].reshape(1, -1),
                mlp[0]['be'].reshape(1, -1),
                wt[1], bt[1], mlp[1]['g'].reshape(1, -1),
                mlp[1]['be'].reshape(1, -1),
                wt[2], bt[2], mlp[2]['g'].reshape(1, -1),
                mlp[2]['be'].reshape(1, -1),
                wf1, bf1, gf1.reshape(1, -1), ef1.reshape(1, -1),
                wf2, bf2)
    fai2 = _run_cfa(cg_all, cfa_args)

    pp1, mp1, vp1, pa1, ma1, va1 = _run_mm2(p3, a3, c3, wp1, bp1, wa1, ba1)
    ap1, cp1 = _bn_affine(mp1, vp1, gp1, ep1)
    aa1, ca1 = _bn_affine(ma1, va1, ga1, ea1)

    alpha2, pg1, mg1, vg1 = _run_pag(pp1, ap1, cp1, wp2, bp2,
                                     pa1, aa1, ca1, wa2, ba2,
                                     fai2, wg1, bg1)
    ag1, cg1 = _bn_affine(mg1, vg1, gg1, eg1)

    y = _run_final(pg1, ag1, cg1, wg2, bg2, alpha2)
    y = y.reshape(_B, _N_CENTER, 256)

    center_xyz = cc[:, :, 0:3]
    center_eula = cc[:, :, 3:6]
    center_near = cc[:, :, 6:8]
    center_meta = cc[:, :, 8:12]
    center_fea = cc[:, :, 12:76]
    new_fea_out = jnp.concatenate([center_fea, y], axis=2)
    return center_xyz, center_eula, center_near, center_meta, new_fea_out


# speedup vs baseline: 1.2291x; 1.1159x over previous
import functools

import jax
import jax.numpy as jnp
from jax.experimental import pallas as pl
from jax.experimental.pallas import tpu as pltpu
from jax.experimental.pallas import tpu_sc as plsc

_N_CENTER = 512
_N_NEAR = 32
_B = 8
_N = 2048
_SBLK = 16
_NSB = _N_CENTER // _SBLK
_BR = 2048
_ROWS = _B * _N_CENTER * _N_NEAR
_CROWS = _B * _N_CENTER
_F_CH = 76
_IN_CH = 82
_EPS = 1e-5


def _dot(a, b):
    return jax.lax.dot_general(a, b, (((1,), (0,)), ((), ())),
                               preferred_element_type=jnp.float32)


def _dotb(a, b):
    return jax.lax.dot_general(a.astype(jnp.bfloat16), b.astype(jnp.bfloat16),
                               (((1,), (0,)), ((), ())),
                               preferred_element_type=jnp.float32)


def _dot_t(a, b):
    return jax.lax.dot_general(a, b, (((1,), (1,)), ((), ())),
                               preferred_element_type=jnp.float32)


def _fps_body(xs_ref, ys_ref, zs_ref, f0_ref, out_ref):
    xs = xs_ref[...]
    ys = ys_ref[...]
    zs = zs_ref[...]
    col = jax.lax.broadcasted_iota(jnp.int32, (_B, _N), 1)
    lane_c = jax.lax.broadcasted_iota(jnp.int32, (_B, _N_CENTER), 1)

    def step(t, carry):
        dist, far, cents = carry
        cents = jnp.where(lane_c == t, far, cents)
        sel = col == far
        cx = jnp.sum(jnp.where(sel, xs, 0.0), axis=1, keepdims=True)
        cy = jnp.sum(jnp.where(sel, ys, 0.0), axis=1, keepdims=True)
        cz = jnp.sum(jnp.where(sel, zs, 0.0), axis=1, keepdims=True)
        dx = xs - cx
        dy = ys - cy
        dz = zs - cz
        d = (dx * dx + dy * dy) + dz * dz
        dist = jnp.minimum(dist, d)
        m = jnp.max(dist, axis=1, keepdims=True)
        far = jnp.min(jnp.where(dist == m, col, _N), axis=1, keepdims=True)
        return dist, far, cents

    dist0 = jnp.full((_B, _N), 1e10, jnp.float32)
    far0 = f0_ref[...]
    cents0 = jnp.zeros((_B, _N_CENTER), jnp.int32)

    def step4(q, carry):
        t0 = q * 4
        for u in range(4):
            carry = step(t0 + u, carry)
        return carry

    _, _, cents = jax.lax.fori_loop(0, _N_CENTER // 4, step4,
                                    (dist0, far0, cents0))
    out_ref[...] = cents


def _run_fps(xs, ys, zs, f0):
    return pl.pallas_call(
        _fps_body,
        out_shape=jax.ShapeDtypeStruct((_B, _N_CENTER), jnp.int32),
    )(xs, ys, zs, f0)


def _knn_body(p_ref, xs_ref, ys_ref, zs_ref, fps_ref, out_ref):
    p = p_ref[0]
    xs = xs_ref[0]
    ys = ys_ref[0]
    zs = zs_ref[0]
    fpsc = fps_ref[0]

    colc = jax.lax.broadcasted_iota(jnp.int32, (_N_CENTER, _N), 1)
    oh = (colc == fpsc).astype(jnp.float32)
    px = p[:, 0:1]
    py = p[:, 1:2]
    pz = p[:, 2:3]
    sqf = (px * px + py * py) + pz * pz
    aug = jnp.concatenate([p, sqf], axis=1)
    cg = _dot(oh, aug)
    c = cg[:, 0:3]
    sqc = cg[:, 3:4]
    sq_row = (xs * xs + ys * ys) + zs * zs
    dots = _dot_t(c, p)
    d = (sqc + sq_row) - 2.0 * dots

    lane_k = jax.lax.broadcasted_iota(jnp.int32, (_N_CENTER, _N_NEAR), 1)

    def step(k, carry):
        d_cur, acc = carry
        m = jnp.min(d_cur, axis=1, keepdims=True)
        j = jnp.min(jnp.where(d_cur == m, colc, _N), axis=1, keepdims=True)
        acc = jnp.where(lane_k == k, j, acc)
        d_cur = jnp.where(colc == j, jnp.inf, d_cur)
        return d_cur, acc

    acc0 = jnp.zeros((_N_CENTER, _N_NEAR), jnp.int32)
    carry = (d, acc0)
    for k in range(_N_NEAR):
        carry = step(k, carry)
    out_ref[0] = carry[1]


def _run_knn(xyz, xs, ys, zs, fps3):
    return pl.pallas_call(
        _knn_body,
        grid=(_B,),
        in_specs=[
            pl.BlockSpec((1, _N, 3), lambda b: (b, 0, 0)),
            pl.BlockSpec((1, 1, _N), lambda b: (b, 0, 0)),
            pl.BlockSpec((1, 1, _N), lambda b: (b, 0, 0)),
            pl.BlockSpec((1, 1, _N), lambda b: (b, 0, 0)),
            pl.BlockSpec((1, _N_CENTER, 1), lambda b: (b, 0, 0)),
        ],
        out_specs=pl.BlockSpec((1, _N_CENTER, _N_NEAR), lambda b: (b, 0, 0)),
        out_shape=jax.ShapeDtypeStruct((_B, _N_CENTER, _N_NEAR), jnp.int32),
    )(xyz, xs, ys, zs, fps3)


_GROWS = _ROWS + _CROWS
_NW = 32
_RPW = _GROWS // _NW
_GCH = 12
_CHUNK = _RPW // _GCH
_PCH = 128


def _run_sc_gather(table, gidx):
    mesh = plsc.VectorSubcoreMesh(core_axis_name="c", subcore_axis_name="s")

    @functools.partial(
        pl.kernel, mesh=mesh,
        out_type=jax.ShapeDtypeStruct((_GROWS, _PCH), jnp.float32),
        scratch_types=[
            pltpu.VMEM((_RPW,), jnp.int32),
            pltpu.VMEM((_CHUNK, _PCH), jnp.float32),
            pltpu.VMEM((_CHUNK, _PCH), jnp.float32),
            pltpu.SemaphoreType.DMA,
            pltpu.SemaphoreType.DMA,
        ],
    )
    def k(table_hbm, idx_hbm, out_hbm, idx_v, buf0, buf1, sem0, sem1):
        wid = jax.lax.axis_index("s") * 2 + jax.lax.axis_index("c")
        base = wid * _RPW
        pltpu.sync_copy(idx_hbm.at[pl.ds(base, _RPW)], idx_v)
        bufs = (buf0, buf1)
        sems = (sem0, sem1)
        cps = [None, None]
        cps[0] = pltpu.async_copy(
            table_hbm.at[idx_v.at[pl.ds(0, _CHUNK)]], bufs[0], sems[0])
        for i in range(_GCH):
            cps[i % 2].wait()
            if i + 1 < _GCH:
                cps[(i + 1) % 2] = pltpu.async_copy(
                    table_hbm.at[idx_v.at[pl.ds((i + 1) * _CHUNK, _CHUNK)]],
                    bufs[(i + 1) % 2], sems[(i + 1) % 2])
            pltpu.sync_copy(bufs[i % 2],
                            out_hbm.at[pl.ds(base + i * _CHUNK, _CHUNK)])

    return k(table, gidx)


def _l1_body(xg_ref, cg_ref, w_ref, b_ref, o_ref, s1_ref, s2_ref):
    xg = xg_ref[0, 0]
    cg = cg_ref[0, 0]

    cb = jnp.broadcast_to(cg.reshape(_SBLK, 1, _PCH),
                          (_SBLK, _N_NEAR, _PCH)).reshape(
                              _SBLK * _N_NEAR, _PCH)
    x0 = jnp.concatenate([
        xg[:, 0:3] - cb[:, 0:3],
        xg[:, 3:6] - cb[:, 3:6],
        xg[:, 6:8], cb[:, 6:8],
        xg[:, 8:12], cb[:, 8:12],
        xg[:, 12:76],
    ], axis=1)
    p = _dotb(x0, w_ref[...]) + b_ref[...]
    o_ref[0, 0] = p.astype(jnp.bfloat16)
    s1_ref[0, 0] = jnp.sum(p, axis=0, keepdims=True)
    s2_ref[0, 0] = jnp.sum(p * p, axis=0, keepdims=True)


def _run_l1(xg4, cg4, w, b2):
    cout = w.shape[1]
    outs = pl.pallas_call(
        _l1_body,
        grid=(_B, _NSB),
        in_specs=[
            pl.BlockSpec((1, 1, _SBLK * _N_NEAR, _PCH),
                         lambda b, s: (b, s, 0, 0)),
            pl.BlockSpec((1, 1, _SBLK, _PCH), lambda b, s: (b, s, 0, 0)),
            pl.BlockSpec((_IN_CH, cout), lambda b, s: (0, 0)),
            pl.BlockSpec((1, cout), lambda b, s: (0, 0)),
        ],
        out_specs=[
            pl.BlockSpec((1, 1, _SBLK * _N_NEAR, cout),
                         lambda b, s: (b, s, 0, 0)),
            pl.BlockSpec((1, 1, 1, cout), lambda b, s: (b, s, 0, 0)),
            pl.BlockSpec((1, 1, 1, cout), lambda b, s: (b, s, 0, 0)),
        ],
        out_shape=[
            jax.ShapeDtypeStruct((_B, _NSB, _SBLK * _N_NEAR, cout),
                                 jnp.bfloat16),
            jax.ShapeDtypeStruct((_B, _NSB, 1, cout), jnp.float32),
            jax.ShapeDtypeStruct((_B, _NSB, 1, cout), jnp.float32),
        ],
    )(xg4, cg4, w, b2)
    p, s1, s2 = outs
    p = p.reshape(_ROWS, cout)
    mean = jnp.sum(s1, axis=(0, 1, 2)) / _ROWS
    var = jnp.sum(s2, axis=(0, 1, 2)) / _ROWS - mean * mean
    return p, mean, var


def _mm_body(aff, relu, stats, x_ref, a_ref, c_ref, w_ref, b_ref,
             o_ref, s1_ref, s2_ref):
    h = x_ref[...].astype(jnp.float32)
    if aff:
        h = h * a_ref[...] + c_ref[...]
    if relu:
        h = jnp.maximum(h, 0.0)
    p = _dotb(h, w_ref[...]) + b_ref[...]
    o_ref[...] = p.astype(jnp.bfloat16)
    if stats:
        s1_ref[0] = jnp.sum(p, axis=0, keepdims=True)
        s2_ref[0] = jnp.sum(p * p, axis=0, keepdims=True)


def _run_mm(x, w, b2, a2=None, c2=None, relu=False, stats=True):
    rows, cin = x.shape
    cout = w.shape[1]
    g = rows // _BR
    aff = a2 is not None
    if not aff:
        a2 = jnp.zeros((1, cin), jnp.float32)
        c2 = jnp.zeros((1, cin), jnp.float32)
    body = functools.partial(_mm_body, aff, relu, stats)
    outs = pl.pallas_call(
        body,
        grid=(g,),
        in_specs=[
            pl.BlockSpec((_BR, cin), lambda i: (i, 0)),
            pl.BlockSpec((1, cin), lambda i: (0, 0)),
            pl.BlockSpec((1, cin), lambda i: (0, 0)),
            pl.BlockSpec((cin, cout), lambda i: (0, 0)),
            pl.BlockSpec((1, cout), lambda i: (0, 0)),
        ],
        out_specs=[
            pl.BlockSpec((_BR, cout), lambda i: (i, 0)),
            pl.BlockSpec((1, 1, cout), lambda i: (i, 0, 0)),
            pl.BlockSpec((1, 1, cout), lambda i: (i, 0, 0)),
        ],
        out_shape=[
            jax.ShapeDtypeStruct((rows, cout), jnp.bfloat16),
            jax.ShapeDtypeStruct((g, 1, cout), jnp.float32),
            jax.ShapeDtypeStruct((g, 1, cout), jnp.float32),
        ],
    )(x, a2, c2, w, b2)
    p, s1, s2 = outs
    if not stats:
        return p, None, None
    mean = jnp.sum(s1, axis=(0, 1)) / rows
    var = jnp.sum(s2, axis=(0, 1)) / rows - mean * mean
    return p, mean, var


def _mm2_body(x_ref, a_ref, c_ref, w1_ref, b1_ref, w2_ref, b2_ref,
              o1_ref, o2_ref, s11_ref, s12_ref, s21_ref, s22_ref):
    h = jnp.maximum(x_ref[...].astype(jnp.float32) * a_ref[...] +
                    c_ref[...], 0.0)
    p1 = _dotb(h, w1_ref[...]) + b1_ref[...]
    p2 = _dotb(h, w2_ref[...]) + b2_ref[...]
    o1_ref[...] = p1.astype(jnp.bfloat16)
    o2_ref[...] = p2.astype(jnp.bfloat16)
    s11_ref[0] = jnp.sum(p1, axis=0, keepdims=True)
    s12_ref[0] = jnp.sum(p1 * p1, axis=0, keepdims=True)
    s21_ref[0] = jnp.sum(p2, axis=0, keepdims=True)
    s22_ref[0] = jnp.sum(p2 * p2, axis=0, keepdims=True)


def _run_mm2(x, a2, c2, w1, b1, w2, b2):
    rows, cin = x.shape
    cout = w1.shape[1]
    g = rows // _BR
    bcast = lambda i: (0, 0)
    outs = pl.pallas_call(
        _mm2_body,
        grid=(g,),
        in_specs=[
            pl.BlockSpec((_BR, cin), lambda i: (i, 0)),
            pl.BlockSpec((1, cin), bcast),
            pl.BlockSpec((1, cin), bcast),
            pl.BlockSpec((cin, cout), bcast),
            pl.BlockSpec((1, cout), bcast),
            pl.BlockSpec((cin, cout), bcast),
            pl.BlockSpec((1, cout), bcast),
        ],
        out_specs=[
            pl.BlockSpec((_BR, cout), lambda i: (i, 0)),
            pl.BlockSpec((_BR, cout), lambda i: (i, 0)),
            pl.BlockSpec((1, 1, cout), lambda i: (i, 0, 0)),
            pl.BlockSpec((1, 1, cout), lambda i: (i, 0, 0)),
            pl.BlockSpec((1, 1, cout), lambda i: (i, 0, 0)),
            pl.BlockSpec((1, 1, cout), lambda i: (i, 0, 0)),
        ],
        out_shape=[
            jax.ShapeDtypeStruct((rows, cout), jnp.bfloat16),
            jax.ShapeDtypeStruct((rows, cout), jnp.bfloat16),
            jax.ShapeDtypeStruct((g, 1, cout), jnp.float32),
            jax.ShapeDtypeStruct((g, 1, cout), jnp.float32),
            jax.ShapeDtypeStruct((g, 1, cout), jnp.float32),
            jax.ShapeDtypeStruct((g, 1, cout), jnp.float32),
        ],
    )(x, a2, c2, w1, b1, w2, b2)
    p1, p2, s11, s12, s21, s22 = outs
    m1 = jnp.sum(s11, axis=(0, 1)) / rows
    v1 = jnp.sum(s12, axis=(0, 1)) / rows - m1 * m1
    m2 = jnp.sum(s21, axis=(0, 1)) / rows
    v2 = jnp.sum(s22, axis=(0, 1)) / rows - m2 * m2
    return p1, m1, v1, p2, m2, v2


def _pag_body(pp_ref, ap_ref, cp_ref, wp2_ref, bp2_ref,
              pa_ref, aa_ref, ca_ref, wa2_ref, ba2_ref,
              fai_ref, wg1_ref, bg1_ref,
              al_ref, pg_ref, s1_ref, s2_ref):
    hp = jnp.maximum(pp_ref[...].astype(jnp.float32) * ap_ref[...] +
                     cp_ref[...], 0.0)
    psi2 = _dotb(hp, wp2_ref[...]) + bp2_ref[...]
    ha = jnp.maximum(pa_ref[...].astype(jnp.float32) * aa_ref[...] +
                     ca_ref[...], 0.0)
    al_ref[...] = (_dotb(ha, wa2_ref[...]) + ba2_ref[...]).astype(jnp.bfloat16)
    gc = _BR // _N_NEAR
    gin = (fai_ref[...].reshape(gc, 1, 256) -
           psi2.reshape(gc, _N_NEAR, 256)).reshape(_BR, 256)
    pg = _dotb(gin, wg1_ref[...]) + bg1_ref[...]
    pg_ref[...] = pg.astype(jnp.bfloat16)
    s1_ref[0] = jnp.sum(pg, axis=0, keepdims=True)
    s2_ref[0] = jnp.sum(pg * pg, axis=0, keepdims=True)


def _run_pag(pp1, ap, cp, wp2, bp2, pa1, aa, ca, wa2, ba2, fai2, wg1, bg1):
    g = _ROWS // _BR
    gc = _BR // _N_NEAR
    bcast2 = lambda i: (0, 0)
    outs = pl.pallas_call(
        _pag_body,
        grid=(g,),
        in_specs=[
            pl.BlockSpec((_BR, 264), lambda i: (i, 0)),
            pl.BlockSpec((1, 264), bcast2),
            pl.BlockSpec((1, 264), bcast2),
            pl.BlockSpec((264, 256), bcast2),
            pl.BlockSpec((1, 256), bcast2),
            pl.BlockSpec((_BR, 264), lambda i: (i, 0)),
            pl.BlockSpec((1, 264), bcast2),
            pl.BlockSpec((1, 264), bcast2),
            pl.BlockSpec((264, 256), bcast2),
            pl.BlockSpec((1, 256), bcast2),
            pl.BlockSpec((gc, 256), lambda i: (i, 0)),
            pl.BlockSpec((256, 264), bcast2),
            pl.BlockSpec((1, 264), bcast2),
        ],
        out_specs=[
            pl.BlockSpec((_BR, 256), lambda i: (i, 0)),
            pl.BlockSpec((_BR, 264), lambda i: (i, 0)),
            pl.BlockSpec((1, 1, 264), lambda i: (i, 0, 0)),
            pl.BlockSpec((1, 1, 264), lambda i: (i, 0, 0)),
        ],
        out_shape=[
            jax.ShapeDtypeStruct((_ROWS, 256), jnp.bfloat16),
            jax.ShapeDtypeStruct((_ROWS, 264), jnp.bfloat16),
            jax.ShapeDtypeStruct((g, 1, 264), jnp.float32),
            jax.ShapeDtypeStruct((g, 1, 264), jnp.float32),
        ],
    )(pp1, ap, cp, wp2, bp2, pa1, aa, ca, wa2, ba2, fai2, wg1, bg1)
    alpha2, pg1, s1, s2 = outs
    mean = jnp.sum(s1, axis=(0, 1)) / _ROWS
    var = jnp.sum(s2, axis=(0, 1)) / _ROWS - mean * mean
    return alpha2, pg1, mean, var


def _bn_affine(mean, var, g, be):
    a = g / jnp.sqrt(var + _EPS)
    c = be - mean * a
    return a.reshape(1, -1), c.reshape(1, -1)


def _cfa_body(cg_ref, w1_ref, b1_ref, g1_ref, e1_ref, w2_ref, b2_ref,
              g2_ref, e2_ref, w3_ref, b3_ref, g3_ref, e3_ref,
              wf1_ref, bf1_ref, gf1_ref, ef1_ref, wf2_ref, bf2_ref,
              out_ref):
    cg = cg_ref[...]
    x = jnp.concatenate([
        jnp.zeros((_CROWS, 6), jnp.float32),
        cg[:, 6:8], cg[:, 6:8],
        cg[:, 8:12], cg[:, 8:12],
        cg[:, 12:76],
    ], axis=1)

    def bnrelu(p, g, e):
        m = jnp.mean(p, axis=0, keepdims=True)
        v = jnp.mean((p - m) * (p - m), axis=0, keepdims=True)
        return jnp.maximum(g * (p - m) / jnp.sqrt(v + _EPS) + e, 0.0)

    x = bnrelu(_dotb(x, w1_ref[...]) + b1_ref[...], g1_ref[...], e1_ref[...])
    x = bnrelu(_dotb(x, w2_ref[...]) + b2_ref[...], g2_ref[...], e2_ref[...])
    x = bnrelu(_dotb(x, w3_ref[...]) + b3_ref[...], g3_ref[...], e3_ref[...])
    x = bnrelu(_dotb(x, wf1_ref[...]) + bf1_ref[...], gf1_ref[...],
               ef1_ref[...])
    out_ref[...] = _dotb(x, wf2_ref[...]) + bf2_ref[...]


def _run_cfa(cg_all, args):
    return pl.pallas_call(
        _cfa_body,
        out_shape=jax.ShapeDtypeStruct((_CROWS, 256), jnp.float32),
    )(cg_all, *args)


def _final_body(pg_ref, a_ref, c_ref, w_ref, b_ref, al_ref, y_ref):
    h = jnp.maximum(pg_ref[...].astype(jnp.float32) * a_ref[...] +
                    c_ref[...], 0.0)
    gam = _dotb(h, w_ref[...]) + b_ref[...]
    m = jnp.max(gam, axis=1, keepdims=True)
    e = jnp.exp(gam - m)
    s = jnp.sum(e, axis=1, keepdims=True)
    contrib = (e / s) * al_ref[...].astype(jnp.float32) * (256.0 / _N_NEAR)
    gc = _BR // _N_NEAR
    y_ref[...] = jnp.sum(contrib.reshape(gc, _N_NEAR, 256), axis=1)


def _run_final(pg1, ag, cg, w, b2, alpha2):
    g = _ROWS // _BR
    gc = _BR // _N_NEAR
    return pl.pallas_call(
        _final_body,
        grid=(g,),
        in_specs=[
            pl.BlockSpec((_BR, 264), lambda i: (i, 0)),
            pl.BlockSpec((1, 264), lambda i: (0, 0)),
            pl.BlockSpec((1, 264), lambda i: (0, 0)),
            pl.BlockSpec((264, 256), lambda i: (0, 0)),
            pl.BlockSpec((1, 256), lambda i: (0, 0)),
            pl.BlockSpec((_BR, 256), lambda i: (i, 0)),
        ],
        out_specs=pl.BlockSpec((gc, 256), lambda i: (i, 0)),
        out_shape=jax.ShapeDtypeStruct((_CROWS, 256), jnp.float32),
    )(pg1, ag, cg, w, b2, alpha2)


@jax.jit
def kernel(xyz, eula_angle, edge_nearby, meta_type, fea, params):
    xs = xyz[:, :, 0]
    ys = xyz[:, :, 1]
    zs = xyz[:, :, 2]
    f0 = jax.random.randint(jax.random.key(42), (_B,), 0, _N,
                            dtype=jnp.int32).reshape(_B, 1)

    fps_idx = _run_fps(xs, ys, zs, f0)
    fps3 = fps_idx.reshape(_B, _N_CENTER, 1)
    xs3 = xs.reshape(_B, 1, _N)
    ys3 = ys.reshape(_B, 1, _N)
    zs3 = zs.reshape(_B, 1, _N)
    idx = _run_knn(xyz, xs3, ys3, zs3, fps3)

    feat = jnp.concatenate([xyz, eula_angle, edge_nearby, meta_type, fea,
                            jnp.zeros((_B, _N, _PCH - _F_CH), jnp.float32)],
                           axis=2)
    boff = (jnp.arange(_B, dtype=jnp.int32) * _N).reshape(_B, 1, 1)
    gidx = jnp.concatenate([
        (idx + boff).reshape(-1),
        (fps_idx + boff[:, :, 0]).reshape(-1),
    ])
    gath = _run_sc_gather(feat.reshape(_B * _N, _PCH), gidx)
    xg4 = gath[:_ROWS].reshape(_B, _NSB, _SBLK * _N_NEAR, _PCH)
    cg_all = gath[_ROWS:]
    cg4 = cg_all.reshape(_B, _NSB, _SBLK, _PCH)
    cc = cg_all.reshape(_B, _N_CENTER, _PCH)

    mlp = params['mlp']
    att = params['att']
    wt = [jnp.transpose(layer['w']) for layer in mlp]
    bt = [layer['b'].reshape(1, -1) for layer in mlp]

    p1, m1, v1 = _run_l1(xg4, cg4, wt[0], bt[0])
    a1, c1 = _bn_affine(m1, v1, mlp[0]['g'], mlp[0]['be'])
    p2, m2, v2 = _run_mm(p1, wt[1], bt[1], a1, c1, relu=True)
    a2, c2 = _bn_affine(m2, v2, mlp[1]['g'], mlp[1]['be'])
    p3, m3, v3 = _run_mm(p2, wt[2], bt[2], a2, c2, relu=True)
    a3, c3 = _bn_affine(m3, v3, mlp[2]['g'], mlp[2]['be'])

    def att_w(name):
        p = att[name]
        return (jnp.transpose(p['w1']), p['b1'].reshape(1, -1), p['g1'],
                p['be1'], jnp.transpose(p['w2']), p['b2'].reshape(1, -1))

    wp1, bp1, gp1, ep1, wp2, bp2 = att_w('psi')
    wa1, ba1, ga1, ea1, wa2, ba2 = att_w('alpha')
    wf1, bf1, gf1, ef1, wf2, bf2 = att_w('fai')
    wg1, bg1, gg1, eg1, wg2, bg2 = att_w('gamma')

    cfa_args = (wt[0], bt[0], mlp[0]['g'].reshape(1, -1),
                mlp[0]['be'].reshape(1, -1),
                wt[1], bt[1], mlp[1]['g'].reshape(1, -1),
                mlp[1]['be'].reshape(1, -1),
                wt[2], bt[2], mlp[2]['g'].reshape(1, -1),
                mlp[2]['be'].reshape(1, -1),
                wf1, bf1, gf1.reshape(1, -1), ef1.reshape(1, -1),
                wf2, bf2)
    fai2 = _run_cfa(cg_all, cfa_args)

    pp1, mp1, vp1, pa1, ma1, va1 = _run_mm2(p3, a3, c3, wp1, bp1, wa1, ba1)
    ap1, cp1 = _bn_affine(mp1, vp1, gp1, ep1)
    aa1, ca1 = _bn_affine(ma1, va1, ga1, ea1)

    alpha2, pg1, mg1, vg1 = _run_pag(pp1, ap1, cp1, wp2, bp2,
                                     pa1, aa1, ca1, wa2, ba2,
                                     fai2, wg1, bg1)
    ag1, cg1 = _bn_affine(mg1, vg1, gg1, eg1)

    y = _run_final(pg1, ag1, cg1, wg2, bg2, alpha2)
    y = y.reshape(_B, _N_CENTER, 256)

    center_xyz = cc[:, :, 0:3]
    center_eula = cc[:, :, 3:6]
    center_near = cc[:, :, 6:8]
    center_meta = cc[:, :, 8:12]
    center_fea = cc[:, :, 12:76]
    new_fea_out = jnp.concatenate([center_fea, y], axis=2)
    return center_xyz, center_eula, center_near, center_meta, new_fea_out


# speedup vs baseline: 1.2988x; 1.0567x over previous
import functools

import jax
import jax.numpy as jnp
from jax.experimental import pallas as pl
from jax.experimental.pallas import tpu as pltpu
from jax.experimental.pallas import tpu_sc as plsc

_N_CENTER = 512
_N_NEAR = 32
_B = 8
_N = 2048
_SBLK = 16
_NSB = _N_CENTER // _SBLK
_BR = 4096
_ROWS = _B * _N_CENTER * _N_NEAR
_CROWS = _B * _N_CENTER
_F_CH = 76
_IN_CH = 82
_EPS = 1e-5


def _dot(a, b):
    return jax.lax.dot_general(a, b, (((1,), (0,)), ((), ())),
                               preferred_element_type=jnp.float32)


def _dotb(a, b):
    return jax.lax.dot_general(a.astype(jnp.bfloat16), b.astype(jnp.bfloat16),
                               (((1,), (0,)), ((), ())),
                               preferred_element_type=jnp.float32)


def _dot_t(a, b):
    return jax.lax.dot_general(a, b, (((1,), (1,)), ((), ())),
                               preferred_element_type=jnp.float32)


def _fps_body(xs_ref, ys_ref, zs_ref, f0_ref, out_ref):
    xs = xs_ref[...]
    ys = ys_ref[...]
    zs = zs_ref[...]
    col = jax.lax.broadcasted_iota(jnp.int32, (_B, _N), 1)
    lane_c = jax.lax.broadcasted_iota(jnp.int32, (_B, _N_CENTER), 1)

    def step(t, carry):
        dist, far, cents = carry
        cents = jnp.where(lane_c == t, far, cents)
        sel = col == far
        cx = jnp.sum(jnp.where(sel, xs, 0.0), axis=1, keepdims=True)
        cy = jnp.sum(jnp.where(sel, ys, 0.0), axis=1, keepdims=True)
        cz = jnp.sum(jnp.where(sel, zs, 0.0), axis=1, keepdims=True)
        dx = xs - cx
        dy = ys - cy
        dz = zs - cz
        d = (dx * dx + dy * dy) + dz * dz
        dist = jnp.minimum(dist, d)
        m = jnp.max(dist, axis=1, keepdims=True)
        far = jnp.min(jnp.where(dist == m, col, _N), axis=1, keepdims=True)
        return dist, far, cents

    dist0 = jnp.full((_B, _N), 1e10, jnp.float32)
    far0 = f0_ref[...]
    cents0 = jnp.zeros((_B, _N_CENTER), jnp.int32)

    def step4(q, carry):
        t0 = q * 4
        for u in range(4):
            carry = step(t0 + u, carry)
        return carry

    _, _, cents = jax.lax.fori_loop(0, _N_CENTER // 4, step4,
                                    (dist0, far0, cents0))
    out_ref[...] = cents


def _run_fps(xs, ys, zs, f0):
    return pl.pallas_call(
        _fps_body,
        out_shape=jax.ShapeDtypeStruct((_B, _N_CENTER), jnp.int32),
    )(xs, ys, zs, f0)


def _knn_body(p_ref, xs_ref, ys_ref, zs_ref, fps_ref, out_ref):
    p = p_ref[0]
    xs = xs_ref[0]
    ys = ys_ref[0]
    zs = zs_ref[0]
    fpsc = fps_ref[0]

    colc = jax.lax.broadcasted_iota(jnp.int32, (_N_CENTER, _N), 1)
    oh = (colc == fpsc).astype(jnp.float32)
    px = p[:, 0:1]
    py = p[:, 1:2]
    pz = p[:, 2:3]
    sqf = (px * px + py * py) + pz * pz
    aug = jnp.concatenate([p, sqf], axis=1)
    cg = _dot(oh, aug)
    c = cg[:, 0:3]
    sqc = cg[:, 3:4]
    sq_row = (xs * xs + ys * ys) + zs * zs
    dots = _dot_t(c, p)
    d = (sqc + sq_row) - 2.0 * dots

    lane_k = jax.lax.broadcasted_iota(jnp.int32, (_N_CENTER, _N_NEAR), 1)

    def step(k, carry):
        d_cur, acc = carry
        m = jnp.min(d_cur, axis=1, keepdims=True)
        j = jnp.min(jnp.where(d_cur == m, colc, _N), axis=1, keepdims=True)
        acc = jnp.where(lane_k == k, j, acc)
        d_cur = jnp.where(colc == j, jnp.inf, d_cur)
        return d_cur, acc

    acc0 = jnp.zeros((_N_CENTER, _N_NEAR), jnp.int32)
    carry = (d, acc0)
    for k in range(_N_NEAR):
        carry = step(k, carry)
    out_ref[0] = carry[1]


def _run_knn(xyz, xs, ys, zs, fps3):
    return pl.pallas_call(
        _knn_body,
        grid=(_B,),
        in_specs=[
            pl.BlockSpec((1, _N, 3), lambda b: (b, 0, 0)),
            pl.BlockSpec((1, 1, _N), lambda b: (b, 0, 0)),
            pl.BlockSpec((1, 1, _N), lambda b: (b, 0, 0)),
            pl.BlockSpec((1, 1, _N), lambda b: (b, 0, 0)),
            pl.BlockSpec((1, _N_CENTER, 1), lambda b: (b, 0, 0)),
        ],
        out_specs=pl.BlockSpec((1, _N_CENTER, _N_NEAR), lambda b: (b, 0, 0)),
        out_shape=jax.ShapeDtypeStruct((_B, _N_CENTER, _N_NEAR), jnp.int32),
    )(xyz, xs, ys, zs, fps3)


_GROWS = _ROWS + _CROWS
_NW = 32
_RPW = _GROWS // _NW
_GCH = 12
_CHUNK = _RPW // _GCH
_PCH = 128


def _run_sc_gather(table, gidx):
    mesh = plsc.VectorSubcoreMesh(core_axis_name="c", subcore_axis_name="s")

    @functools.partial(
        pl.kernel, mesh=mesh,
        out_type=jax.ShapeDtypeStruct((_GROWS, _PCH), jnp.float32),
        scratch_types=[
            pltpu.VMEM((_RPW,), jnp.int32),
            pltpu.VMEM((_CHUNK, _PCH), jnp.float32),
            pltpu.VMEM((_CHUNK, _PCH), jnp.float32),
            pltpu.SemaphoreType.DMA,
            pltpu.SemaphoreType.DMA,
        ],
    )
    def k(table_hbm, idx_hbm, out_hbm, idx_v, buf0, buf1, sem0, sem1):
        wid = jax.lax.axis_index("s") * 2 + jax.lax.axis_index("c")
        base = wid * _RPW
        pltpu.sync_copy(idx_hbm.at[pl.ds(base, _RPW)], idx_v)
        bufs = (buf0, buf1)
        sems = (sem0, sem1)
        cps = [None, None]
        cps[0] = pltpu.async_copy(
            table_hbm.at[idx_v.at[pl.ds(0, _CHUNK)]], bufs[0], sems[0])
        for i in range(_GCH):
            cps[i % 2].wait()
            if i + 1 < _GCH:
                cps[(i + 1) % 2] = pltpu.async_copy(
                    table_hbm.at[idx_v.at[pl.ds((i + 1) * _CHUNK, _CHUNK)]],
                    bufs[(i + 1) % 2], sems[(i + 1) % 2])
            pltpu.sync_copy(bufs[i % 2],
                            out_hbm.at[pl.ds(base + i * _CHUNK, _CHUNK)])

    return k(table, gidx)


def _l1_body(xg_ref, cg_ref, w_ref, b_ref, o_ref, s1_ref, s2_ref):
    xg = xg_ref[0, 0]
    cg = cg_ref[0, 0]

    cb = jnp.broadcast_to(cg.reshape(_SBLK, 1, _PCH),
                          (_SBLK, _N_NEAR, _PCH)).reshape(
                              _SBLK * _N_NEAR, _PCH)
    x0 = jnp.concatenate([
        xg[:, 0:3] - cb[:, 0:3],
        xg[:, 3:6] - cb[:, 3:6],
        xg[:, 6:8], cb[:, 6:8],
        xg[:, 8:12], cb[:, 8:12],
        xg[:, 12:76],
    ], axis=1)
    p = _dotb(x0, w_ref[...]) + b_ref[...]
    o_ref[0, 0] = p.astype(jnp.bfloat16)
    s1_ref[0, 0] = jnp.sum(p, axis=0, keepdims=True)
    s2_ref[0, 0] = jnp.sum(p * p, axis=0, keepdims=True)


def _run_l1(xg4, cg4, w, b2):
    cout = w.shape[1]
    outs = pl.pallas_call(
        _l1_body,
        grid=(_B, _NSB),
        in_specs=[
            pl.BlockSpec((1, 1, _SBLK * _N_NEAR, _PCH),
                         lambda b, s: (b, s, 0, 0)),
            pl.BlockSpec((1, 1, _SBLK, _PCH), lambda b, s: (b, s, 0, 0)),
            pl.BlockSpec((_IN_CH, cout), lambda b, s: (0, 0)),
            pl.BlockSpec((1, cout), lambda b, s: (0, 0)),
        ],
        out_specs=[
            pl.BlockSpec((1, 1, _SBLK * _N_NEAR, cout),
                         lambda b, s: (b, s, 0, 0)),
            pl.BlockSpec((1, 1, 1, cout), lambda b, s: (b, s, 0, 0)),
            pl.BlockSpec((1, 1, 1, cout), lambda b, s: (b, s, 0, 0)),
        ],
        out_shape=[
            jax.ShapeDtypeStruct((_B, _NSB, _SBLK * _N_NEAR, cout),
                                 jnp.bfloat16),
            jax.ShapeDtypeStruct((_B, _NSB, 1, cout), jnp.float32),
            jax.ShapeDtypeStruct((_B, _NSB, 1, cout), jnp.float32),
        ],
    )(xg4, cg4, w, b2)
    p, s1, s2 = outs
    p = p.reshape(_ROWS, cout)
    mean = jnp.sum(s1, axis=(0, 1, 2)) / _ROWS
    var = jnp.sum(s2, axis=(0, 1, 2)) / _ROWS - mean * mean
    return p, mean, var


def _mm_body(aff, relu, stats, x_ref, a_ref, c_ref, w_ref, b_ref,
             o_ref, s1_ref, s2_ref):
    h = x_ref[...].astype(jnp.float32)
    if aff:
        h = h * a_ref[...] + c_ref[...]
    if relu:
        h = jnp.maximum(h, 0.0)
    p = _dotb(h, w_ref[...]) + b_ref[...]
    o_ref[...] = p.astype(jnp.bfloat16)
    if stats:
        s1_ref[0] = jnp.sum(p, axis=0, keepdims=True)
        s2_ref[0] = jnp.sum(p * p, axis=0, keepdims=True)


def _run_mm(x, w, b2, a2=None, c2=None, relu=False, stats=True):
    rows, cin = x.shape
    cout = w.shape[1]
    g = rows // _BR
    aff = a2 is not None
    if not aff:
        a2 = jnp.zeros((1, cin), jnp.float32)
        c2 = jnp.zeros((1, cin), jnp.float32)
    body = functools.partial(_mm_body, aff, relu, stats)
    outs = pl.pallas_call(
        body,
        grid=(g,),
        in_specs=[
            pl.BlockSpec((_BR, cin), lambda i: (i, 0)),
            pl.BlockSpec((1, cin), lambda i: (0, 0)),
            pl.BlockSpec((1, cin), lambda i: (0, 0)),
            pl.BlockSpec((cin, cout), lambda i: (0, 0)),
            pl.BlockSpec((1, cout), lambda i: (0, 0)),
        ],
        out_specs=[
            pl.BlockSpec((_BR, cout), lambda i: (i, 0)),
            pl.BlockSpec((1, 1, cout), lambda i: (i, 0, 0)),
            pl.BlockSpec((1, 1, cout), lambda i: (i, 0, 0)),
        ],
        out_shape=[
            jax.ShapeDtypeStruct((rows, cout), jnp.bfloat16),
            jax.ShapeDtypeStruct((g, 1, cout), jnp.float32),
            jax.ShapeDtypeStruct((g, 1, cout), jnp.float32),
        ],
    )(x, a2, c2, w, b2)
    p, s1, s2 = outs
    if not stats:
        return p, None, None
    mean = jnp.sum(s1, axis=(0, 1)) / rows
    var = jnp.sum(s2, axis=(0, 1)) / rows - mean * mean
    return p, mean, var


def _mm2_body(x_ref, a_ref, c_ref, w1_ref, b1_ref, w2_ref, b2_ref,
              o1_ref, o2_ref, s11_ref, s12_ref, s21_ref, s22_ref):
    h = jnp.maximum(x_ref[...].astype(jnp.float32) * a_ref[...] +
                    c_ref[...], 0.0)
    p1 = _dotb(h, w1_ref[...]) + b1_ref[...]
    p2 = _dotb(h, w2_ref[...]) + b2_ref[...]
    o1_ref[...] = p1.astype(jnp.bfloat16)
    o2_ref[...] = p2.astype(jnp.bfloat16)
    s11_ref[0] = jnp.sum(p1, axis=0, keepdims=True)
    s12_ref[0] = jnp.sum(p1 * p1, axis=0, keepdims=True)
    s21_ref[0] = jnp.sum(p2, axis=0, keepdims=True)
    s22_ref[0] = jnp.sum(p2 * p2, axis=0, keepdims=True)


def _run_mm2(x, a2, c2, w1, b1, w2, b2):
    rows, cin = x.shape
    cout = w1.shape[1]
    g = rows // _BR
    bcast = lambda i: (0, 0)
    outs = pl.pallas_call(
        _mm2_body,
        grid=(g,),
        in_specs=[
            pl.BlockSpec((_BR, cin), lambda i: (i, 0)),
            pl.BlockSpec((1, cin), bcast),
            pl.BlockSpec((1, cin), bcast),
            pl.BlockSpec((cin, cout), bcast),
            pl.BlockSpec((1, cout), bcast),
            pl.BlockSpec((cin, cout), bcast),
            pl.BlockSpec((1, cout), bcast),
        ],
        out_specs=[
            pl.BlockSpec((_BR, cout), lambda i: (i, 0)),
            pl.BlockSpec((_BR, cout), lambda i: (i, 0)),
            pl.BlockSpec((1, 1, cout), lambda i: (i, 0, 0)),
            pl.BlockSpec((1, 1, cout), lambda i: (i, 0, 0)),
            pl.BlockSpec((1, 1, cout), lambda i: (i, 0, 0)),
            pl.BlockSpec((1, 1, cout), lambda i: (i, 0, 0)),
        ],
        out_shape=[
            jax.ShapeDtypeStruct((rows, cout), jnp.bfloat16),
            jax.ShapeDtypeStruct((rows, cout), jnp.bfloat16),
            jax.ShapeDtypeStruct((g, 1, cout), jnp.float32),
            jax.ShapeDtypeStruct((g, 1, cout), jnp.float32),
            jax.ShapeDtypeStruct((g, 1, cout), jnp.float32),
            jax.ShapeDtypeStruct((g, 1, cout), jnp.float32),
        ],
    )(x, a2, c2, w1, b1, w2, b2)
    p1, p2, s11, s12, s21, s22 = outs
    m1 = jnp.sum(s11, axis=(0, 1)) / rows
    v1 = jnp.sum(s12, axis=(0, 1)) / rows - m1 * m1
    m2 = jnp.sum(s21, axis=(0, 1)) / rows
    v2 = jnp.sum(s22, axis=(0, 1)) / rows - m2 * m2
    return p1, m1, v1, p2, m2, v2


def _pag_body(pp_ref, ap_ref, cp_ref, wp2_ref, bp2_ref,
              pa_ref, aa_ref, ca_ref, wa2_ref, ba2_ref,
              fai_ref, wg1_ref, bg1_ref,
              al_ref, pg_ref, s1_ref, s2_ref):
    hp = jnp.maximum(pp_ref[...].astype(jnp.float32) * ap_ref[...] +
                     cp_ref[...], 0.0)
    psi2 = _dotb(hp, wp2_ref[...]) + bp2_ref[...]
    ha = jnp.maximum(pa_ref[...].astype(jnp.float32) * aa_ref[...] +
                     ca_ref[...], 0.0)
    al_ref[...] = (_dotb(ha, wa2_ref[...]) + ba2_ref[...]).astype(jnp.bfloat16)
    gc = _BR // _N_NEAR
    gin = (fai_ref[...].reshape(gc, 1, 256) -
           psi2.reshape(gc, _N_NEAR, 256)).reshape(_BR, 256)
    pg = _dotb(gin, wg1_ref[...]) + bg1_ref[...]
    pg_ref[...] = pg.astype(jnp.bfloat16)
    s1_ref[0] = jnp.sum(pg, axis=0, keepdims=True)
    s2_ref[0] = jnp.sum(pg * pg, axis=0, keepdims=True)


def _run_pag(pp1, ap, cp, wp2, bp2, pa1, aa, ca, wa2, ba2, fai2, wg1, bg1):
    g = _ROWS // _BR
    gc = _BR // _N_NEAR
    bcast2 = lambda i: (0, 0)
    outs = pl.pallas_call(
        _pag_body,
        grid=(g,),
        in_specs=[
            pl.BlockSpec((_BR, 264), lambda i: (i, 0)),
            pl.BlockSpec((1, 264), bcast2),
            pl.BlockSpec((1, 264), bcast2),
            pl.BlockSpec((264, 256), bcast2),
            pl.BlockSpec((1, 256), bcast2),
            pl.BlockSpec((_BR, 264), lambda i: (i, 0)),
            pl.BlockSpec((1, 264), bcast2),
            pl.BlockSpec((1, 264), bcast2),
            pl.BlockSpec((264, 256), bcast2),
            pl.BlockSpec((1, 256), bcast2),
            pl.BlockSpec((gc, 256), lambda i: (i, 0)),
            pl.BlockSpec((256, 264), bcast2),
            pl.BlockSpec((1, 264), bcast2),
        ],
        out_specs=[
            pl.BlockSpec((_BR, 256), lambda i: (i, 0)),
            pl.BlockSpec((_BR, 264), lambda i: (i, 0)),
            pl.BlockSpec((1, 1, 264), lambda i: (i, 0, 0)),
            pl.BlockSpec((1, 1, 264), lambda i: (i, 0, 0)),
        ],
        out_shape=[
            jax.ShapeDtypeStruct((_ROWS, 256), jnp.bfloat16),
            jax.ShapeDtypeStruct((_ROWS, 264), jnp.bfloat16),
            jax.ShapeDtypeStruct((g, 1, 264), jnp.float32),
            jax.ShapeDtypeStruct((g, 1, 264), jnp.float32),
        ],
    )(pp1, ap, cp, wp2, bp2, pa1, aa, ca, wa2, ba2, fai2, wg1, bg1)
    alpha2, pg1, s1, s2 = outs
    mean = jnp.sum(s1, axis=(0, 1)) / _ROWS
    var = jnp.sum(s2, axis=(0, 1)) / _ROWS - mean * mean
    return alpha2, pg1, mean, var


def _bn_affine(mean, var, g, be):
    a = g / jnp.sqrt(var + _EPS)
    c = be - mean * a
    return a.reshape(1, -1), c.reshape(1, -1)


def _cfa_body(cg_ref, w1_ref, b1_ref, g1_ref, e1_ref, w2_ref, b2_ref,
              g2_ref, e2_ref, w3_ref, b3_ref, g3_ref, e3_ref,
              wf1_ref, bf1_ref, gf1_ref, ef1_ref, wf2_ref, bf2_ref,
              out_ref):
    cg = cg_ref[...]
    x = jnp.concatenate([
        jnp.zeros((_CROWS, 6), jnp.float32),
        cg[:, 6:8], cg[:, 6:8],
        cg[:, 8:12], cg[:, 8:12],
        cg[:, 12:76],
    ], axis=1)

    def bnrelu(p, g, e):
        m = jnp.mean(p, axis=0, keepdims=True)
        v = jnp.mean((p - m) * (p - m), axis=0, keepdims=True)
        return jnp.maximum(g * (p - m) / jnp.sqrt(v + _EPS) + e, 0.0)

    x = bnrelu(_dotb(x, w1_ref[...]) + b1_ref[...], g1_ref[...], e1_ref[...])
    x = bnrelu(_dotb(x, w2_ref[...]) + b2_ref[...], g2_ref[...], e2_ref[...])
    x = bnrelu(_dotb(x, w3_ref[...]) + b3_ref[...], g3_ref[...], e3_ref[...])
    x = bnrelu(_dotb(x, wf1_ref[...]) + bf1_ref[...], gf1_ref[...],
               ef1_ref[...])
    out_ref[...] = _dotb(x, wf2_ref[...]) + bf2_ref[...]


def _run_cfa(cg_all, args):
    return pl.pallas_call(
        _cfa_body,
        out_shape=jax.ShapeDtypeStruct((_CROWS, 256), jnp.float32),
    )(cg_all, *args)


def _final_body(pg_ref, a_ref, c_ref, w_ref, b_ref, al_ref, y_ref):
    h = jnp.maximum(pg_ref[...].astype(jnp.float32) * a_ref[...] +
                    c_ref[...], 0.0)
    gam = _dotb(h, w_ref[...]) + b_ref[...]
    m = jnp.max(gam, axis=1, keepdims=True)
    e = jnp.exp(gam - m)
    s = jnp.sum(e, axis=1, keepdims=True)
    contrib = (e / s) * al_ref[...].astype(jnp.float32) * (256.0 / _N_NEAR)
    gc = _BR // _N_NEAR
    y_ref[...] = jnp.sum(contrib.reshape(gc, _N_NEAR, 256), axis=1)


def _run_final(pg1, ag, cg, w, b2, alpha2):
    g = _ROWS // _BR
    gc = _BR // _N_NEAR
    return pl.pallas_call(
        _final_body,
        grid=(g,),
        in_specs=[
            pl.BlockSpec((_BR, 264), lambda i: (i, 0)),
            pl.BlockSpec((1, 264), lambda i: (0, 0)),
            pl.BlockSpec((1, 264), lambda i: (0, 0)),
            pl.BlockSpec((264, 256), lambda i: (0, 0)),
            pl.BlockSpec((1, 256), lambda i: (0, 0)),
            pl.BlockSpec((_BR, 256), lambda i: (i, 0)),
        ],
        out_specs=pl.BlockSpec((gc, 256), lambda i: (i, 0)),
        out_shape=jax.ShapeDtypeStruct((_CROWS, 256), jnp.float32),
    )(pg1, ag, cg, w, b2, alpha2)


@jax.jit
def kernel(xyz, eula_angle, edge_nearby, meta_type, fea, params):
    xs = xyz[:, :, 0]
    ys = xyz[:, :, 1]
    zs = xyz[:, :, 2]
    f0 = jax.random.randint(jax.random.key(42), (_B,), 0, _N,
                            dtype=jnp.int32).reshape(_B, 1)

    fps_idx = _run_fps(xs, ys, zs, f0)
    fps3 = fps_idx.reshape(_B, _N_CENTER, 1)
    xs3 = xs.reshape(_B, 1, _N)
    ys3 = ys.reshape(_B, 1, _N)
    zs3 = zs.reshape(_B, 1, _N)
    idx = _run_knn(xyz, xs3, ys3, zs3, fps3)

    feat = jnp.concatenate([xyz, eula_angle, edge_nearby, meta_type, fea,
                            jnp.zeros((_B, _N, _PCH - _F_CH), jnp.float32)],
                           axis=2)
    boff = (jnp.arange(_B, dtype=jnp.int32) * _N).reshape(_B, 1, 1)
    gidx = jnp.concatenate([
        (idx + boff).reshape(-1),
        (fps_idx + boff[:, :, 0]).reshape(-1),
    ])
    gath = _run_sc_gather(feat.reshape(_B * _N, _PCH), gidx)
    xg4 = gath[:_ROWS].reshape(_B, _NSB, _SBLK * _N_NEAR, _PCH)
    cg_all = gath[_ROWS:]
    cg4 = cg_all.reshape(_B, _NSB, _SBLK, _PCH)
    cc = cg_all.reshape(_B, _N_CENTER, _PCH)

    mlp = params['mlp']
    att = params['att']
    wt = [jnp.transpose(layer['w']) for layer in mlp]
    bt = [layer['b'].reshape(1, -1) for layer in mlp]

    p1, m1, v1 = _run_l1(xg4, cg4, wt[0], bt[0])
    a1, c1 = _bn_affine(m1, v1, mlp[0]['g'], mlp[0]['be'])
    p2, m2, v2 = _run_mm(p1, wt[1], bt[1], a1, c1, relu=True)
    a2, c2 = _bn_affine(m2, v2, mlp[1]['g'], mlp[1]['be'])
    p3, m3, v3 = _run_mm(p2, wt[2], bt[2], a2, c2, relu=True)
    a3, c3 = _bn_affine(m3, v3, mlp[2]['g'], mlp[2]['be'])

    def att_w(name):
        p = att[name]
        return (jnp.transpose(p['w1']), p['b1'].reshape(1, -1), p['g1'],
                p['be1'], jnp.transpose(p['w2']), p['b2'].reshape(1, -1))

    wp1, bp1, gp1, ep1, wp2, bp2 = att_w('psi')
    wa1, ba1, ga1, ea1, wa2, ba2 = att_w('alpha')
    wf1, bf1, gf1, ef1, wf2, bf2 = att_w('fai')
    wg1, bg1, gg1, eg1, wg2, bg2 = att_w('gamma')

    cfa_args = (wt[0], bt[0], mlp[0]['g'].reshape(1, -1),
                mlp[0]['be'].reshape(1, -1),
                wt[1], bt[1], mlp[1]['g'].reshape(1, -1),
                mlp[1]['be'].reshape(1, -1),
                wt[2], bt[2], mlp[2]['g'].reshape(1, -1),
                mlp[2]['be'].reshape(1, -1),
                wf1, bf1, gf1.reshape(1, -1), ef1.reshape(1, -1),
                wf2, bf2)
    fai2 = _run_cfa(cg_all, cfa_args)

    pp1, mp1, vp1, pa1, ma1, va1 = _run_mm2(p3, a3, c3, wp1, bp1, wa1, ba1)
    ap1, cp1 = _bn_affine(mp1, vp1, gp1, ep1)
    aa1, ca1 = _bn_affine(ma1, va1, ga1, ea1)

    alpha2, pg1, mg1, vg1 = _run_pag(pp1, ap1, cp1, wp2, bp2,
                                     pa1, aa1, ca1, wa2, ba2,
                                     fai2, wg1, bg1)
    ag1, cg1 = _bn_affine(mg1, vg1, gg1, eg1)

    y = _run_final(pg1, ag1, cg1, wg2, bg2, alpha2)
    y = y.reshape(_B, _N_CENTER, 256)

    center_xyz = cc[:, :, 0:3]
    center_eula = cc[:, :, 3:6]
    center_near = cc[:, :, 6:8]
    center_meta = cc[:, :, 8:12]
    center_fea = cc[:, :, 12:76]
    new_fea_out = jnp.concatenate([center_fea, y], axis=2)
    return center_xyz, center_eula, center_near, center_meta, new_fea_out
